# baseline (speedup 1.0000x reference)
import functools

import jax
import jax.numpy as jnp
from jax.experimental import pallas as pl
from jax.experimental.pallas import tpu as pltpu

_SLOPE = 0.01
_KC = 4


def _ceil_to(a, b):
    return (a + b - 1) // b * b


def _leaky(v):
    return jnp.where(v >= 0, v, _SLOPE * v)


def _sigmoid(v):
    z = jnp.exp(-jnp.abs(v))
    return jnp.where(v >= 0, 1.0 / (1.0 + z), z / (1.0 + z))


def _params(dims):
    return pltpu.CompilerParams(dimension_semantics=dims,
                                vmem_limit_bytes=56 * 1024 * 1024)


def _msg_body(ea_ref, xs_ref, w1_ref, b1_ref, w2_ref, o_ref, h_sc, acc_sc,
              *, kc, out_ch, nk):
    k = pl.program_id(1)

    @pl.when(k == 0)
    def _():
        h = jnp.dot(ea_ref[...], w1_ref[...],
                    preferred_element_type=jnp.float32) + b1_ref[...]
        h_sc[...] = _leaky(h)
        acc_sc[...] = jnp.zeros_like(acc_sc)

    xsk = xs_ref[0]
    h = h_sc[...]
    part = jnp.dot(xsk[:, 0:1] * h, w2_ref[:, 0:out_ch],
                   preferred_element_type=jnp.float32)
    for kk in range(1, kc):
        part = part + jnp.dot(
            xsk[:, kk:kk + 1] * h, w2_ref[:, kk * out_ch:(kk + 1) * out_ch],
            preferred_element_type=jnp.float32)
    acc_sc[...] += part

    @pl.when(k == nk - 1)
    def _():
        o_ref[...] = acc_sc[...]


def _messages(ea8, xs_chunks, w2, w1p, b1, out_ch, te):
    nk, e_pad, _ = xs_chunks.shape
    hdim = w1p.shape[1]
    return pl.pallas_call(
        functools.partial(_msg_body, kc=_KC, out_ch=out_ch, nk=nk),
        out_shape=jax.ShapeDtypeStruct((e_pad, out_ch), jnp.float32),
        grid=(e_pad // te, nk),
        in_specs=[
            pl.BlockSpec((te, 8), lambda e, k: (e, 0)),
            pl.BlockSpec((1, te, _KC), lambda e, k: (k, e, 0)),
            pl.BlockSpec((8, hdim), lambda e, k: (0, 0)),
            pl.BlockSpec((1, hdim), lambda e, k: (0, 0)),
            pl.BlockSpec((hdim, _KC * out_ch), lambda e, k: (0, k)),
        ],
        out_specs=pl.BlockSpec((te, out_ch), lambda e, k: (e, 0)),
        scratch_shapes=[pltpu.VMEM((te, hdim), jnp.float32),
                        pltpu.VMEM((te, out_ch), jnp.float32)],
        compiler_params=_params(("parallel", "arbitrary")),
    )(ea8, xs_chunks, w1p, b1, w2)


def _agg_body(tgt_ref, m_ref, xs_ref, x_ref, wr_ref, b2_ref, b_ref, o_ref,
              acc_sc, sx_sc, deg_sc, *, tn):
    n = pl.program_id(0)
    e = pl.program_id(1)

    @pl.when(e == 0)
    def _():
        acc_sc[...] = jnp.zeros_like(acc_sc)
        sx_sc[...] = jnp.zeros_like(sx_sc)
        deg_sc[...] = jnp.zeros_like(deg_sc)

    tec = m_ref.shape[0]
    rows = (jax.lax.broadcasted_iota(jnp.int32, (tn, tec), 0)
            + n * tn).astype(jnp.float32)
    mask = (rows == tgt_ref[...]).astype(jnp.float32)
    acc_sc[...] += jnp.dot(mask, m_ref[...],
                           preferred_element_type=jnp.float32)
    sx_sc[...] += jnp.dot(mask, xs_ref[...],
                          preferred_element_type=jnp.float32)
    deg_sc[...] += jnp.sum(mask, axis=1, keepdims=True)

    @pl.when(e == pl.num_programs(1) - 1)
    def _():
        agg = acc_sc[...] + jnp.dot(sx_sc[...], b2_ref[...],
                                    preferred_element_type=jnp.float32)
        root = jnp.dot(x_ref[...], wr_ref[...],
                       preferred_element_type=jnp.float32)
        inv = 1.0 / jnp.maximum(deg_sc[...], 1.0)
        o_ref[...] = _leaky(agg * inv + root + b_ref[...])


def _combine(tgtf, msgs, xs_src, x_nodes, w_root, b2_mat, bias, tn, tec):
    n_pad = x_nodes.shape[0]
    e_pad = msgs.shape[0]
    out_ch = msgs.shape[1]
    kdim = x_nodes.shape[1]
    return pl.pallas_call(
        functools.partial(_agg_body, tn=tn),
        out_shape=jax.ShapeDtypeStruct((n_pad, out_ch), jnp.float32),
        grid=(n_pad // tn, e_pad // tec),
        in_specs=[
            pl.BlockSpec((1, tec), lambda n, e: (0, e)),
            pl.BlockSpec((tec, out_ch), lambda n, e: (e, 0)),
            pl.BlockSpec((tec, kdim), lambda n, e: (e, 0)),
            pl.BlockSpec((tn, kdim), lambda n, e: (n, 0)),
            pl.BlockSpec((kdim, out_ch), lambda n, e: (0, 0)),
            pl.BlockSpec((kdim, out_ch), lambda n, e: (0, 0)),
            pl.BlockSpec((1, out_ch), lambda n, e: (0, 0)),
        ],
        out_specs=pl.BlockSpec((tn, out_ch), lambda n, e: (n, 0)),
        scratch_shapes=[pltpu.VMEM((tn, out_ch), jnp.float32),
                        pltpu.VMEM((tn, kdim), jnp.float32),
                        pltpu.VMEM((tn, 1), jnp.float32)],
        compiler_params=_params(("parallel", "arbitrary")),
    )(tgtf, msgs, xs_src, x_nodes, w_root, b2_mat, bias)


def _head_body(bf_ref, d_ref, x_ref, w1d_ref, w1x_ref, b1_ref,
               w2_ref, b2_ref, w3_ref, b3_ref, o_ref,
               pd_sc, px_sc, cnt_sc, *, nb):
    n = pl.program_id(0)

    @pl.when(n == 0)
    def _():
        pd_sc[...] = jnp.zeros_like(pd_sc)
        px_sc[...] = jnp.zeros_like(px_sc)
        cnt_sc[...] = jnp.zeros_like(cnt_sc)

    tn = d_ref.shape[0]
    gids = jax.lax.broadcasted_iota(jnp.int32, (nb, tn), 0).astype(jnp.float32)
    mask = (gids == bf_ref[...]).astype(jnp.float32)
    pd_sc[...] += jnp.dot(mask, d_ref[...],
                          preferred_element_type=jnp.float32)
    px_sc[...] += jnp.dot(mask, x_ref[...],
                          preferred_element_type=jnp.float32)
    cnt_sc[...] += jnp.sum(mask, axis=1, keepdims=True)

    @pl.when(n == pl.num_programs(0) - 1)
    def _():
        inv = 1.0 / jnp.maximum(cnt_sc[...], 1.0)
        h = jnp.dot(pd_sc[...] * inv, w1d_ref[...],
                    preferred_element_type=jnp.float32) \
            + jnp.dot(px_sc[...] * inv, w1x_ref[...],
                      preferred_element_type=jnp.float32) + b1_ref[...]
        h = _leaky(h)
        h = _leaky(jnp.dot(h, w2_ref[...],
                           preferred_element_type=jnp.float32) + b2_ref[...])
        y = jnp.dot(h, w3_ref[...],
                    preferred_element_type=jnp.float32) + b3_ref[...]
        o_ref[...] = _sigmoid(y)


def _head(batchf, d3, x8, w1d, w1x, b1, w2, b2, w3, b3, nb, tn):
    n_pad, ddim = d3.shape
    h1 = w1d.shape[1]
    h2 = w2.shape[1]
    return pl.pallas_call(
        functools.partial(_head_body, nb=nb),
        out_shape=jax.ShapeDtypeStruct((nb, 1), jnp.float32),
        grid=(n_pad // tn,),
        in_specs=[
            pl.BlockSpec((1, tn), lambda n: (0, n)),
            pl.BlockSpec((tn, ddim), lambda n: (n, 0)),
            pl.BlockSpec((tn, 8), lambda n: (n, 0)),
            pl.BlockSpec((ddim, h1), lambda n: (0, 0)),
            pl.BlockSpec((8, h1), lambda n: (0, 0)),
            pl.BlockSpec((1, h1), lambda n: (0, 0)),
            pl.BlockSpec((h1, h2), lambda n: (0, 0)),
            pl.BlockSpec((1, h2), lambda n: (0, 0)),
            pl.BlockSpec((h2, 1), lambda n: (0, 0)),
            pl.BlockSpec((1, 1), lambda n: (0, 0)),
        ],
        out_specs=pl.BlockSpec((nb, 1), lambda n: (0, 0)),
        scratch_shapes=[pltpu.VMEM((nb, ddim), jnp.float32),
                        pltpu.VMEM((nb, 8), jnp.float32),
                        pltpu.VMEM((nb, 1), jnp.float32)],
        compiler_params=_params(("arbitrary",)),
    )(batchf, d3, x8, w1d, w1x, b1, w2, b2, w3, b3)


def _chunked(xs, e_pad):
    kdim = xs.shape[1]
    return xs.reshape(e_pad, kdim // _KC, _KC).transpose(1, 0, 2)


def kernel(x, edge_index, edge_attr, batch,
           conv1_nn_w1, conv1_nn_b1, conv1_nn_w2, conv1_nn_b2,
           conv1_root_w, conv1_bias,
           conv2_nn_w1, conv2_nn_b1, conv2_nn_w2, conv2_nn_b2,
           conv2_root_w, conv2_bias,
           conv3_nn_w1, conv3_nn_b1, conv3_nn_w2, conv3_nn_b2,
           conv3_root_w, conv3_bias,
           fc1_w, fc1_b, fc2_w, fc2_b, fc3_w, fc3_b):
    num_graphs = 64
    x = x.astype(jnp.float32)
    n_nodes, fdim = x.shape
    n_edges = edge_index.shape[1]

    te = 2048 if n_edges % 2048 == 0 else _ceil_to(n_edges, 128)
    e_pad = _ceil_to(n_edges, te)
    tec = te
    n_pad = _ceil_to(n_nodes, 8)
    tn = 512 if n_pad % 512 == 0 else n_pad
    nb = _ceil_to(num_graphs, 8)

    src = edge_index[0]
    tgtf = jnp.full((1, e_pad), -1.0, jnp.float32).at[0, :n_edges].set(
        edge_index[1].astype(jnp.float32))
    batchf = jnp.full((1, n_pad), -1.0, jnp.float32).at[0, :n_nodes].set(
        batch.astype(jnp.float32))

    ea8 = jnp.zeros((e_pad, 8), jnp.float32).at[:n_edges, :4].set(
        edge_attr.astype(jnp.float32))
    x_pad = jnp.zeros((n_pad, fdim), jnp.float32).at[:n_nodes].set(x)
    x8 = jnp.zeros((n_pad, 8), jnp.float32).at[:n_nodes, :4].set(x)

    def conv(d_nodes, w1, b1, w2, b2, w_root, bias, out_ch):
        kdim = d_nodes.shape[1]
        xs_src = d_nodes[src]
        if e_pad > n_edges:
            xs_src = jnp.concatenate(
                [xs_src, jnp.zeros((e_pad - n_edges, kdim), jnp.float32)],
                axis=0)
        xs_chunks = _chunked(xs_src, e_pad)
        w1p = jnp.zeros((8, w1.shape[1]), jnp.float32).at[:w1.shape[0]].set(
            w1.astype(jnp.float32))
        msgs = _messages(ea8, xs_chunks, w2.astype(jnp.float32),
                         w1p, b1.reshape(1, -1).astype(jnp.float32),
                         out_ch, te)
        return _combine(tgtf, msgs, xs_src, d_nodes,
                        w_root.astype(jnp.float32),
                        b2.astype(jnp.float32).reshape(kdim, out_ch),
                        bias.reshape(1, -1).astype(jnp.float32), tn, tec)

    c1 = conv(x_pad, conv1_nn_w1, conv1_nn_b1, conv1_nn_w2, conv1_nn_b2,
              conv1_root_w, conv1_bias, 256)
    d1 = jnp.concatenate([c1, x_pad], axis=1)
    c2 = conv(d1, conv2_nn_w1, conv2_nn_b1, conv2_nn_w2, conv2_nn_b2,
              conv2_root_w, conv2_bias, 256)
    d2 = jnp.concatenate([c2, x_pad], axis=1)
    c3 = conv(d2, conv3_nn_w1, conv3_nn_b1, conv3_nn_w2, conv3_nn_b2,
              conv3_root_w, conv3_bias, 512)

    ddim = c3.shape[1]
    w1d = fc1_w[:ddim].astype(jnp.float32)
    w1x = jnp.zeros((8, fc1_w.shape[1]), jnp.float32).at[:fdim].set(
        fc1_w[ddim:].astype(jnp.float32))
    out = _head(batchf, c3, x8, w1d, w1x,
                fc1_b.reshape(1, -1).astype(jnp.float32),
                fc2_w.astype(jnp.float32),
                fc2_b.reshape(1, -1).astype(jnp.float32),
                fc3_w.astype(jnp.float32),
                fc3_b.reshape(1, -1).astype(jnp.float32), nb, tn)
    return out[:num_graphs]


# speedup vs baseline: 1.3482x; 1.3482x over previous
import functools

import jax
import jax.numpy as jnp
from jax.experimental import pallas as pl
from jax.experimental.pallas import tpu as pltpu

_SLOPE = 0.01
_KC = 4


def _ceil_to(a, b):
    return (a + b - 1) // b * b


def _leaky(v):
    return jnp.where(v >= 0, v, _SLOPE * v)


def _sigmoid(v):
    z = jnp.exp(-jnp.abs(v))
    return jnp.where(v >= 0, 1.0 / (1.0 + z), z / (1.0 + z))


def _params(dims):
    return pltpu.CompilerParams(dimension_semantics=dims,
                                vmem_limit_bytes=56 * 1024 * 1024)


def _msg_body(ea_ref, xs_ref, w1_ref, b1_ref, w2_ref, o_ref, h_sc, acc_sc,
              *, kc, out_ch, nk, kdim):
    k = pl.program_id(1)

    @pl.when(k == 0)
    def _():
        h = jnp.dot(ea_ref[...], w1_ref[...],
                    preferred_element_type=jnp.float32) + b1_ref[...]
        h_sc[...] = _leaky(h)
        acc_sc[...] = jnp.zeros_like(acc_sc)

    rows = jax.lax.broadcasted_iota(jnp.int32, (kdim, kc), 0)
    cols = jax.lax.broadcasted_iota(jnp.int32, (kdim, kc), 1)
    sel = (rows == k * kc + cols).astype(jnp.float32)
    xsk = jnp.dot(xs_ref[...], sel, preferred_element_type=jnp.float32)

    h = h_sc[...]
    part = jnp.dot(xsk[:, 0:1] * h, w2_ref[:, 0:out_ch],
                   preferred_element_type=jnp.float32)
    for kk in range(1, kc):
        part = part + jnp.dot(
            xsk[:, kk:kk + 1] * h, w2_ref[:, kk * out_ch:(kk + 1) * out_ch],
            preferred_element_type=jnp.float32)
    acc_sc[...] += part

    @pl.when(k == nk - 1)
    def _():
        o_ref[...] = acc_sc[...]


def _messages(ea8, xs_src, w2, w1p, b1, out_ch, te):
    e_pad, kdim = xs_src.shape
    nk = kdim // _KC
    hdim = w1p.shape[1]
    return pl.pallas_call(
        functools.partial(_msg_body, kc=_KC, out_ch=out_ch, nk=nk, kdim=kdim),
        out_shape=jax.ShapeDtypeStruct((e_pad, out_ch), jnp.float32),
        grid=(e_pad // te, nk),
        in_specs=[
            pl.BlockSpec((te, 8), lambda e, k: (e, 0)),
            pl.BlockSpec((te, kdim), lambda e, k: (e, 0)),
            pl.BlockSpec((8, hdim), lambda e, k: (0, 0)),
            pl.BlockSpec((1, hdim), lambda e, k: (0, 0)),
            pl.BlockSpec((hdim, _KC * out_ch), lambda e, k: (0, k)),
        ],
        out_specs=pl.BlockSpec((te, out_ch), lambda e, k: (e, 0)),
        scratch_shapes=[pltpu.VMEM((te, hdim), jnp.float32),
                        pltpu.VMEM((te, out_ch), jnp.float32)],
        compiler_params=_params(("parallel", "arbitrary")),
    )(ea8, xs_src, w1p, b1, w2)


def _agg_body(tgt_ref, m_ref, xs_ref, x_ref, wr_ref, b2_ref, b_ref, o_ref,
              acc_sc, sx_sc, deg_sc, *, tn):
    n = pl.program_id(0)
    e = pl.program_id(1)

    @pl.when(e == 0)
    def _():
        acc_sc[...] = jnp.zeros_like(acc_sc)
        sx_sc[...] = jnp.zeros_like(sx_sc)
        deg_sc[...] = jnp.zeros_like(deg_sc)

    tec = m_ref.shape[0]
    rows = (jax.lax.broadcasted_iota(jnp.int32, (tn, tec), 0)
            + n * tn).astype(jnp.float32)
    mask = (rows == tgt_ref[...]).astype(jnp.float32)
    acc_sc[...] += jnp.dot(mask, m_ref[...],
                           preferred_element_type=jnp.float32)
    sx_sc[...] += jnp.dot(mask, xs_ref[...],
                          preferred_element_type=jnp.float32)
    deg_sc[...] += jnp.sum(mask, axis=1, keepdims=True)

    @pl.when(e == pl.num_programs(1) - 1)
    def _():
        agg = acc_sc[...] + jnp.dot(sx_sc[...], b2_ref[...],
                                    preferred_element_type=jnp.float32)
        root = jnp.dot(x_ref[...], wr_ref[...],
                       preferred_element_type=jnp.float32)
        inv = 1.0 / jnp.maximum(deg_sc[...], 1.0)
        o_ref[...] = _leaky(agg * inv + root + b_ref[...])


def _combine(tgtf, msgs, xs_src, x_nodes, w_root, b2_mat, bias, tn, tec):
    n_pad = x_nodes.shape[0]
    e_pad = msgs.shape[0]
    out_ch = msgs.shape[1]
    kdim = x_nodes.shape[1]
    return pl.pallas_call(
        functools.partial(_agg_body, tn=tn),
        out_shape=jax.ShapeDtypeStruct((n_pad, out_ch), jnp.float32),
        grid=(n_pad // tn, e_pad // tec),
        in_specs=[
            pl.BlockSpec((1, tec), lambda n, e: (0, e)),
            pl.BlockSpec((tec, out_ch), lambda n, e: (e, 0)),
            pl.BlockSpec((tec, kdim), lambda n, e: (e, 0)),
            pl.BlockSpec((tn, kdim), lambda n, e: (n, 0)),
            pl.BlockSpec((kdim, out_ch), lambda n, e: (0, 0)),
            pl.BlockSpec((kdim, out_ch), lambda n, e: (0, 0)),
            pl.BlockSpec((1, out_ch), lambda n, e: (0, 0)),
        ],
        out_specs=pl.BlockSpec((tn, out_ch), lambda n, e: (n, 0)),
        scratch_shapes=[pltpu.VMEM((tn, out_ch), jnp.float32),
                        pltpu.VMEM((tn, kdim), jnp.float32),
                        pltpu.VMEM((tn, 1), jnp.float32)],
        compiler_params=_params(("parallel", "arbitrary")),
    )(tgtf, msgs, xs_src, x_nodes, w_root, b2_mat, bias)


def _head_body(bf_ref, d_ref, x_ref, w1d_ref, w1x_ref, b1_ref,
               w2_ref, b2_ref, w3_ref, b3_ref, o_ref,
               pd_sc, px_sc, cnt_sc, *, nb):
    n = pl.program_id(0)

    @pl.when(n == 0)
    def _():
        pd_sc[...] = jnp.zeros_like(pd_sc)
        px_sc[...] = jnp.zeros_like(px_sc)
        cnt_sc[...] = jnp.zeros_like(cnt_sc)

    tn = d_ref.shape[0]
    gids = jax.lax.broadcasted_iota(jnp.int32, (nb, tn), 0).astype(jnp.float32)
    mask = (gids == bf_ref[...]).astype(jnp.float32)
    pd_sc[...] += jnp.dot(mask, d_ref[...],
                          preferred_element_type=jnp.float32)
    px_sc[...] += jnp.dot(mask, x_ref[...],
                          preferred_element_type=jnp.float32)
    cnt_sc[...] += jnp.sum(mask, axis=1, keepdims=True)

    @pl.when(n == pl.num_programs(0) - 1)
    def _():
        inv = 1.0 / jnp.maximum(cnt_sc[...], 1.0)
        h = jnp.dot(pd_sc[...] * inv, w1d_ref[...],
                    preferred_element_type=jnp.float32) \
            + jnp.dot(px_sc[...] * inv, w1x_ref[...],
                      preferred_element_type=jnp.float32) + b1_ref[...]
        h = _leaky(h)
        h = _leaky(jnp.dot(h, w2_ref[...],
                           preferred_element_type=jnp.float32) + b2_ref[...])
        y = jnp.dot(h, w3_ref[...],
                    preferred_element_type=jnp.float32) + b3_ref[...]
        o_ref[...] = _sigmoid(y)


def _head(batchf, d3, x8, w1d, w1x, b1, w2, b2, w3, b3, nb, tn):
    n_pad, ddim = d3.shape
    h1 = w1d.shape[1]
    h2 = w2.shape[1]
    return pl.pallas_call(
        functools.partial(_head_body, nb=nb),
        out_shape=jax.ShapeDtypeStruct((nb, 1), jnp.float32),
        grid=(n_pad // tn,),
        in_specs=[
            pl.BlockSpec((1, tn), lambda n: (0, n)),
            pl.BlockSpec((tn, ddim), lambda n: (n, 0)),
            pl.BlockSpec((tn, 8), lambda n: (n, 0)),
            pl.BlockSpec((ddim, h1), lambda n: (0, 0)),
            pl.BlockSpec((8, h1), lambda n: (0, 0)),
            pl.BlockSpec((1, h1), lambda n: (0, 0)),
            pl.BlockSpec((h1, h2), lambda n: (0, 0)),
            pl.BlockSpec((1, h2), lambda n: (0, 0)),
            pl.BlockSpec((h2, 1), lambda n: (0, 0)),
            pl.BlockSpec((1, 1), lambda n: (0, 0)),
        ],
        out_specs=pl.BlockSpec((nb, 1), lambda n: (0, 0)),
        scratch_shapes=[pltpu.VMEM((nb, ddim), jnp.float32),
                        pltpu.VMEM((nb, 8), jnp.float32),
                        pltpu.VMEM((nb, 1), jnp.float32)],
        compiler_params=_params(("arbitrary",)),
    )(batchf, d3, x8, w1d, w1x, b1, w2, b2, w3, b3)


def kernel(x, edge_index, edge_attr, batch,
           conv1_nn_w1, conv1_nn_b1, conv1_nn_w2, conv1_nn_b2,
           conv1_root_w, conv1_bias,
           conv2_nn_w1, conv2_nn_b1, conv2_nn_w2, conv2_nn_b2,
           conv2_root_w, conv2_bias,
           conv3_nn_w1, conv3_nn_b1, conv3_nn_w2, conv3_nn_b2,
           conv3_root_w, conv3_bias,
           fc1_w, fc1_b, fc2_w, fc2_b, fc3_w, fc3_b):
    num_graphs = 64
    x = x.astype(jnp.float32)
    n_nodes, fdim = x.shape
    n_edges = edge_index.shape[1]

    te = 2048 if n_edges % 2048 == 0 else _ceil_to(n_edges, 128)
    e_pad = _ceil_to(n_edges, te)
    tec = te
    n_pad = _ceil_to(n_nodes, 8)
    tn = 512 if n_pad % 512 == 0 else n_pad
    nb = _ceil_to(num_graphs, 8)

    src = edge_index[0]
    tgtf = jnp.full((1, e_pad), -1.0, jnp.float32).at[0, :n_edges].set(
        edge_index[1].astype(jnp.float32))
    batchf = jnp.full((1, n_pad), -1.0, jnp.float32).at[0, :n_nodes].set(
        batch.astype(jnp.float32))

    ea8 = jnp.zeros((e_pad, 8), jnp.float32).at[:n_edges, :4].set(
        edge_attr.astype(jnp.float32))
    x_pad = jnp.zeros((n_pad, fdim), jnp.float32).at[:n_nodes].set(x)
    x8 = jnp.zeros((n_pad, 8), jnp.float32).at[:n_nodes, :4].set(x)

    def conv(d_nodes, w1, b1, w2, b2, w_root, bias, out_ch):
        kdim = d_nodes.shape[1]
        xs_src = d_nodes[src]
        if e_pad > n_edges:
            xs_src = jnp.concatenate(
                [xs_src, jnp.zeros((e_pad - n_edges, kdim), jnp.float32)],
                axis=0)
        w1p = jnp.zeros((8, w1.shape[1]), jnp.float32).at[:w1.shape[0]].set(
            w1.astype(jnp.float32))
        msgs = _messages(ea8, xs_src, w2.astype(jnp.float32),
                         w1p, b1.reshape(1, -1).astype(jnp.float32),
                         out_ch, te)
        return _combine(tgtf, msgs, xs_src, d_nodes,
                        w_root.astype(jnp.float32),
                        b2.astype(jnp.float32).reshape(kdim, out_ch),
                        bias.reshape(1, -1).astype(jnp.float32), tn, tec)

    c1 = conv(x_pad, conv1_nn_w1, conv1_nn_b1, conv1_nn_w2, conv1_nn_b2,
              conv1_root_w, conv1_bias, 256)
    d1 = jnp.concatenate([c1, x_pad], axis=1)
    c2 = conv(d1, conv2_nn_w1, conv2_nn_b1, conv2_nn_w2, conv2_nn_b2,
              conv2_root_w, conv2_bias, 256)
    d2 = jnp.concatenate([c2, x_pad], axis=1)
    c3 = conv(d2, conv3_nn_w1, conv3_nn_b1, conv3_nn_w2, conv3_nn_b2,
              conv3_root_w, conv3_bias, 512)

    ddim = c3.shape[1]
    w1d = fc1_w[:ddim].astype(jnp.float32)
    w1x = jnp.zeros((8, fc1_w.shape[1]), jnp.float32).at[:fdim].set(
        fc1_w[ddim:].astype(jnp.float32))
    out = _head(batchf, c3, x8, w1d, w1x,
                fc1_b.reshape(1, -1).astype(jnp.float32),
                fc2_w.astype(jnp.float32),
                fc2_b.reshape(1, -1).astype(jnp.float32),
                fc3_w.astype(jnp.float32),
                fc3_b.reshape(1, -1).astype(jnp.float32), nb, tn)
    return out[:num_graphs]


# speedup vs baseline: 1.3627x; 1.0107x over previous
import functools

import jax
import jax.numpy as jnp
from jax.experimental import pallas as pl
from jax.experimental.pallas import tpu as pltpu

_SLOPE = 0.01
_KC = 4


def _ceil_to(a, b):
    return (a + b - 1) // b * b


def _leaky(v):
    return jnp.where(v >= 0, v, _SLOPE * v)


def _sigmoid(v):
    z = jnp.exp(-jnp.abs(v))
    return jnp.where(v >= 0, 1.0 / (1.0 + z), z / (1.0 + z))


def _params(dims):
    return pltpu.CompilerParams(dimension_semantics=dims,
                                vmem_limit_bytes=56 * 1024 * 1024)


def _msg_body(ea_ref, src_ref, d_ref, w1_ref, b1_ref, w2_ref,
              o_ref, oxs_ref, h_sc, xs_sc, acc_sc, *, kc, out_ch, nk, kdim):
    k = pl.program_id(1)
    te = ea_ref.shape[0]
    n_pad = d_ref.shape[0]

    @pl.when(k == 0)
    def _():
        h = jnp.dot(ea_ref[...], w1_ref[...],
                    preferred_element_type=jnp.float32) + b1_ref[...]
        h_sc[...] = _leaky(h)
        node = jax.lax.broadcasted_iota(jnp.int32, (te, n_pad), 1)
        g = (src_ref[...] == node.astype(jnp.float32)).astype(jnp.float32)
        xs = jnp.dot(g, d_ref[...], preferred_element_type=jnp.float32)
        xs_sc[...] = xs
        oxs_ref[...] = xs
        acc_sc[...] = jnp.zeros_like(acc_sc)

    rows = jax.lax.broadcasted_iota(jnp.int32, (kdim, kc), 0)
    cols = jax.lax.broadcasted_iota(jnp.int32, (kdim, kc), 1)
    sel = (rows == k * kc + cols).astype(jnp.float32)
    xsk = jnp.dot(xs_sc[...], sel, preferred_element_type=jnp.float32)

    h = h_sc[...]
    part = jnp.dot(xsk[:, 0:1] * h, w2_ref[:, 0:out_ch],
                   preferred_element_type=jnp.float32)
    for kk in range(1, kc):
        part = part + jnp.dot(
            xsk[:, kk:kk + 1] * h, w2_ref[:, kk * out_ch:(kk + 1) * out_ch],
            preferred_element_type=jnp.float32)
    acc_sc[...] += part

    @pl.when(k == nk - 1)
    def _():
        o_ref[...] = acc_sc[...]


def _messages(ea8, srcf, d_nodes, w2, w1p, b1, out_ch, te):
    e_pad = ea8.shape[0]
    n_pad, kdim = d_nodes.shape
    nk = kdim // _KC
    hdim = w1p.shape[1]
    return pl.pallas_call(
        functools.partial(_msg_body, kc=_KC, out_ch=out_ch, nk=nk, kdim=kdim),
        out_shape=(jax.ShapeDtypeStruct((e_pad, out_ch), jnp.float32),
                   jax.ShapeDtypeStruct((e_pad, kdim), jnp.float32)),
        grid=(e_pad // te, nk),
        in_specs=[
            pl.BlockSpec((te, 8), lambda e, k: (e, 0)),
            pl.BlockSpec((te, 1), lambda e, k: (e, 0)),
            pl.BlockSpec((n_pad, kdim), lambda e, k: (0, 0)),
            pl.BlockSpec((8, hdim), lambda e, k: (0, 0)),
            pl.BlockSpec((1, hdim), lambda e, k: (0, 0)),
            pl.BlockSpec((hdim, _KC * out_ch), lambda e, k: (0, k)),
        ],
        out_specs=(pl.BlockSpec((te, out_ch), lambda e, k: (e, 0)),
                   pl.BlockSpec((te, kdim), lambda e, k: (e, 0))),
        scratch_shapes=[pltpu.VMEM((te, hdim), jnp.float32),
                        pltpu.VMEM((te, kdim), jnp.float32),
                        pltpu.VMEM((te, out_ch), jnp.float32)],
        compiler_params=_params(("parallel", "arbitrary")),
    )(ea8, srcf, d_nodes, w1p, b1, w2)


def _agg_body(tgt_ref, m_ref, xs_ref, x_ref, wr_ref, b2_ref, b_ref, o_ref,
              acc_sc, sx_sc, deg_sc, *, tn):
    n = pl.program_id(0)
    e = pl.program_id(1)

    @pl.when(e == 0)
    def _():
        acc_sc[...] = jnp.zeros_like(acc_sc)
        sx_sc[...] = jnp.zeros_like(sx_sc)
        deg_sc[...] = jnp.zeros_like(deg_sc)

    tec = m_ref.shape[0]
    rows = (jax.lax.broadcasted_iota(jnp.int32, (tn, tec), 0)
            + n * tn).astype(jnp.float32)
    mask = (rows == tgt_ref[...]).astype(jnp.float32)
    acc_sc[...] += jnp.dot(mask, m_ref[...],
                           preferred_element_type=jnp.float32)
    sx_sc[...] += jnp.dot(mask, xs_ref[...],
                          preferred_element_type=jnp.float32)
    deg_sc[...] += jnp.sum(mask, axis=1, keepdims=True)

    @pl.when(e == pl.num_programs(1) - 1)
    def _():
        agg = acc_sc[...] + jnp.dot(sx_sc[...], b2_ref[...],
                                    preferred_element_type=jnp.float32)
        root = jnp.dot(x_ref[...], wr_ref[...],
                       preferred_element_type=jnp.float32)
        inv = 1.0 / jnp.maximum(deg_sc[...], 1.0)
        o_ref[...] = _leaky(agg * inv + root + b_ref[...])


def _combine(tgtf, msgs, xs_src, x_nodes, w_root, b2_mat, bias, tn, tec):
    n_pad = x_nodes.shape[0]
    e_pad = msgs.shape[0]
    out_ch = msgs.shape[1]
    kdim = x_nodes.shape[1]
    return pl.pallas_call(
        functools.partial(_agg_body, tn=tn),
        out_shape=jax.ShapeDtypeStruct((n_pad, out_ch), jnp.float32),
        grid=(n_pad // tn, e_pad // tec),
        in_specs=[
            pl.BlockSpec((1, tec), lambda n, e: (0, e)),
            pl.BlockSpec((tec, out_ch), lambda n, e: (e, 0)),
            pl.BlockSpec((tec, kdim), lambda n, e: (e, 0)),
            pl.BlockSpec((tn, kdim), lambda n, e: (n, 0)),
            pl.BlockSpec((kdim, out_ch), lambda n, e: (0, 0)),
            pl.BlockSpec((kdim, out_ch), lambda n, e: (0, 0)),
            pl.BlockSpec((1, out_ch), lambda n, e: (0, 0)),
        ],
        out_specs=pl.BlockSpec((tn, out_ch), lambda n, e: (n, 0)),
        scratch_shapes=[pltpu.VMEM((tn, out_ch), jnp.float32),
                        pltpu.VMEM((tn, kdim), jnp.float32),
                        pltpu.VMEM((tn, 1), jnp.float32)],
        compiler_params=_params(("parallel", "arbitrary")),
    )(tgtf, msgs, xs_src, x_nodes, w_root, b2_mat, bias)


def _head_body(bf_ref, d_ref, x_ref, w1d_ref, w1x_ref, b1_ref,
               w2_ref, b2_ref, w3_ref, b3_ref, o_ref,
               pd_sc, px_sc, cnt_sc, *, nb):
    n = pl.program_id(0)

    @pl.when(n == 0)
    def _():
        pd_sc[...] = jnp.zeros_like(pd_sc)
        px_sc[...] = jnp.zeros_like(px_sc)
        cnt_sc[...] = jnp.zeros_like(cnt_sc)

    tn = d_ref.shape[0]
    gids = jax.lax.broadcasted_iota(jnp.int32, (nb, tn), 0).astype(jnp.float32)
    mask = (gids == bf_ref[...]).astype(jnp.float32)
    pd_sc[...] += jnp.dot(mask, d_ref[...],
                          preferred_element_type=jnp.float32)
    px_sc[...] += jnp.dot(mask, x_ref[...],
                          preferred_element_type=jnp.float32)
    cnt_sc[...] += jnp.sum(mask, axis=1, keepdims=True)

    @pl.when(n == pl.num_programs(0) - 1)
    def _():
        inv = 1.0 / jnp.maximum(cnt_sc[...], 1.0)
        h = jnp.dot(pd_sc[...] * inv, w1d_ref[...],
                    preferred_element_type=jnp.float32) \
            + jnp.dot(px_sc[...] * inv, w1x_ref[...],
                      preferred_element_type=jnp.float32) + b1_ref[...]
        h = _leaky(h)
        h = _leaky(jnp.dot(h, w2_ref[...],
                           preferred_element_type=jnp.float32) + b2_ref[...])
        y = jnp.dot(h, w3_ref[...],
                    preferred_element_type=jnp.float32) + b3_ref[...]
        o_ref[...] = _sigmoid(y)


def _head(batchf, d3, x8, w1d, w1x, b1, w2, b2, w3, b3, nb, tn):
    n_pad, ddim = d3.shape
    h1 = w1d.shape[1]
    h2 = w2.shape[1]
    return pl.pallas_call(
        functools.partial(_head_body, nb=nb),
        out_shape=jax.ShapeDtypeStruct((nb, 1), jnp.float32),
        grid=(n_pad // tn,),
        in_specs=[
            pl.BlockSpec((1, tn), lambda n: (0, n)),
            pl.BlockSpec((tn, ddim), lambda n: (n, 0)),
            pl.BlockSpec((tn, 8), lambda n: (n, 0)),
            pl.BlockSpec((ddim, h1), lambda n: (0, 0)),
            pl.BlockSpec((8, h1), lambda n: (0, 0)),
            pl.BlockSpec((1, h1), lambda n: (0, 0)),
            pl.BlockSpec((h1, h2), lambda n: (0, 0)),
            pl.BlockSpec((1, h2), lambda n: (0, 0)),
            pl.BlockSpec((h2, 1), lambda n: (0, 0)),
            pl.BlockSpec((1, 1), lambda n: (0, 0)),
        ],
        out_specs=pl.BlockSpec((nb, 1), lambda n: (0, 0)),
        scratch_shapes=[pltpu.VMEM((nb, ddim), jnp.float32),
                        pltpu.VMEM((nb, 8), jnp.float32),
                        pltpu.VMEM((nb, 1), jnp.float32)],
        compiler_params=_params(("arbitrary",)),
    )(batchf, d3, x8, w1d, w1x, b1, w2, b2, w3, b3)


def kernel(x, edge_index, edge_attr, batch,
           conv1_nn_w1, conv1_nn_b1, conv1_nn_w2, conv1_nn_b2,
           conv1_root_w, conv1_bias,
           conv2_nn_w1, conv2_nn_b1, conv2_nn_w2, conv2_nn_b2,
           conv2_root_w, conv2_bias,
           conv3_nn_w1, conv3_nn_b1, conv3_nn_w2, conv3_nn_b2,
           conv3_root_w, conv3_bias,
           fc1_w, fc1_b, fc2_w, fc2_b, fc3_w, fc3_b):
    num_graphs = 64
    x = x.astype(jnp.float32)
    n_nodes, fdim = x.shape
    n_edges = edge_index.shape[1]

    te = 2048 if n_edges % 2048 == 0 else _ceil_to(n_edges, 128)
    e_pad = _ceil_to(n_edges, te)
    tec = te
    n_pad = _ceil_to(n_nodes, 8)
    tn = 512 if n_pad % 512 == 0 else n_pad
    nb = _ceil_to(num_graphs, 8)

    srcf = jnp.full((e_pad, 1), -1.0, jnp.float32).at[:n_edges, 0].set(
        edge_index[0].astype(jnp.float32))
    tgtf = jnp.full((1, e_pad), -1.0, jnp.float32).at[0, :n_edges].set(
        edge_index[1].astype(jnp.float32))
    batchf = jnp.full((1, n_pad), -1.0, jnp.float32).at[0, :n_nodes].set(
        batch.astype(jnp.float32))

    ea8 = jnp.zeros((e_pad, 8), jnp.float32).at[:n_edges, :4].set(
        edge_attr.astype(jnp.float32))
    x_pad = jnp.zeros((n_pad, fdim), jnp.float32).at[:n_nodes].set(x)
    x8 = jnp.zeros((n_pad, 8), jnp.float32).at[:n_nodes, :4].set(x)

    def conv(d_nodes, w1, b1, w2, b2, w_root, bias, out_ch):
        kdim = d_nodes.shape[1]
        w1p = jnp.zeros((8, w1.shape[1]), jnp.float32).at[:w1.shape[0]].set(
            w1.astype(jnp.float32))
        msgs, xs_src = _messages(ea8, srcf, d_nodes, w2.astype(jnp.float32),
                                 w1p, b1.reshape(1, -1).astype(jnp.float32),
                                 out_ch, te)
        return _combine(tgtf, msgs, xs_src, d_nodes,
                        w_root.astype(jnp.float32),
                        b2.astype(jnp.float32).reshape(kdim, out_ch),
                        bias.reshape(1, -1).astype(jnp.float32), tn, tec)

    c1 = conv(x_pad, conv1_nn_w1, conv1_nn_b1, conv1_nn_w2, conv1_nn_b2,
              conv1_root_w, conv1_bias, 256)
    d1 = jnp.concatenate([c1, x_pad], axis=1)
    c2 = conv(d1, conv2_nn_w1, conv2_nn_b1, conv2_nn_w2, conv2_nn_b2,
              conv2_root_w, conv2_bias, 256)
    d2 = jnp.concatenate([c2, x_pad], axis=1)
    c3 = conv(d2, conv3_nn_w1, conv3_nn_b1, conv3_nn_w2, conv3_nn_b2,
              conv3_root_w, conv3_bias, 512)

    ddim = c3.shape[1]
    w1d = fc1_w[:ddim].astype(jnp.float32)
    w1x = jnp.zeros((8, fc1_w.shape[1]), jnp.float32).at[:fdim].set(
        fc1_w[ddim:].astype(jnp.float32))
    out = _head(batchf, c3, x8, w1d, w1x,
                fc1_b.reshape(1, -1).astype(jnp.float32),
                fc2_w.astype(jnp.float32),
                fc2_b.reshape(1, -1).astype(jnp.float32),
                fc3_w.astype(jnp.float32),
                fc3_b.reshape(1, -1).astype(jnp.float32), nb, tn)
    return out[:num_graphs]


# speedup vs baseline: 1.3677x; 1.0037x over previous
import functools

import numpy as np

import jax
import jax.numpy as jnp
from jax.experimental import pallas as pl
from jax.experimental.pallas import tpu as pltpu

try:
    from jax.experimental.shard_map import shard_map
except ImportError:
    from jax import shard_map

_SLOPE = 0.01
_KC = 4


def _ceil_to(a, b):
    return (a + b - 1) // b * b


def _leaky(v):
    return jnp.where(v >= 0, v, _SLOPE * v)


def _sigmoid(v):
    z = jnp.exp(-jnp.abs(v))
    return jnp.where(v >= 0, 1.0 / (1.0 + z), z / (1.0 + z))


def _params(dims):
    return pltpu.CompilerParams(dimension_semantics=dims,
                                vmem_limit_bytes=56 * 1024 * 1024)


def _msg_body(ea_ref, src_ref, d_ref, w1_ref, b1_ref, w2_ref,
              o_ref, oxs_ref, h_sc, xs_sc, acc_sc, *, kc, out_ch, nk, kdim):
    k = pl.program_id(1)
    te = ea_ref.shape[0]
    n_pad = d_ref.shape[0]

    @pl.when(k == 0)
    def _():
        h = jnp.dot(ea_ref[...], w1_ref[...],
                    preferred_element_type=jnp.float32) + b1_ref[...]
        h_sc[...] = _leaky(h)
        node = jax.lax.broadcasted_iota(jnp.int32, (te, n_pad), 1)
        g = (src_ref[...] == node.astype(jnp.float32)).astype(jnp.float32)
        xs = jnp.dot(g, d_ref[...], preferred_element_type=jnp.float32)
        xs_sc[...] = xs
        oxs_ref[...] = xs
        acc_sc[...] = jnp.zeros_like(acc_sc)

    rows = jax.lax.broadcasted_iota(jnp.int32, (kdim, kc), 0)
    cols = jax.lax.broadcasted_iota(jnp.int32, (kdim, kc), 1)
    sel = (rows == k * kc + cols).astype(jnp.float32)
    xsk = jnp.dot(xs_sc[...], sel, preferred_element_type=jnp.float32)

    h = h_sc[...]
    part = jnp.dot(xsk[:, 0:1] * h, w2_ref[:, 0:out_ch],
                   preferred_element_type=jnp.float32)
    for kk in range(1, kc):
        part = part + jnp.dot(
            xsk[:, kk:kk + 1] * h, w2_ref[:, kk * out_ch:(kk + 1) * out_ch],
            preferred_element_type=jnp.float32)
    acc_sc[...] += part

    @pl.when(k == nk - 1)
    def _():
        o_ref[...] = acc_sc[...]


def _messages(ea8, srcf, d_nodes, w2, w1p, b1, out_ch, te):
    e_pad = ea8.shape[0]
    n_pad, kdim = d_nodes.shape
    nk = kdim // _KC
    hdim = w1p.shape[1]
    return pl.pallas_call(
        functools.partial(_msg_body, kc=_KC, out_ch=out_ch, nk=nk, kdim=kdim),
        out_shape=(jax.ShapeDtypeStruct((e_pad, out_ch), jnp.float32),
                   jax.ShapeDtypeStruct((e_pad, kdim), jnp.float32)),
        grid=(e_pad // te, nk),
        in_specs=[
            pl.BlockSpec((te, 8), lambda e, k: (e, 0)),
            pl.BlockSpec((te, 1), lambda e, k: (e, 0)),
            pl.BlockSpec((n_pad, kdim), lambda e, k: (0, 0)),
            pl.BlockSpec((8, hdim), lambda e, k: (0, 0)),
            pl.BlockSpec((1, hdim), lambda e, k: (0, 0)),
            pl.BlockSpec((hdim, _KC * out_ch), lambda e, k: (0, k)),
        ],
        out_specs=(pl.BlockSpec((te, out_ch), lambda e, k: (e, 0)),
                   pl.BlockSpec((te, kdim), lambda e, k: (e, 0))),
        scratch_shapes=[pltpu.VMEM((te, hdim), jnp.float32),
                        pltpu.VMEM((te, kdim), jnp.float32),
                        pltpu.VMEM((te, out_ch), jnp.float32)],
        compiler_params=_params(("parallel", "arbitrary")),
    )(ea8, srcf, d_nodes, w1p, b1, w2)


def _agg_body(tgt_ref, m_ref, xs_ref, oa_ref, os_ref, od_ref,
              acc_sc, sx_sc, deg_sc, *, tn):
    n = pl.program_id(0)
    e = pl.program_id(1)

    @pl.when(e == 0)
    def _():
        acc_sc[...] = jnp.zeros_like(acc_sc)
        sx_sc[...] = jnp.zeros_like(sx_sc)
        deg_sc[...] = jnp.zeros_like(deg_sc)

    tec = m_ref.shape[0]
    rows = (jax.lax.broadcasted_iota(jnp.int32, (tn, tec), 0)
            + n * tn).astype(jnp.float32)
    mask = (rows == tgt_ref[...]).astype(jnp.float32)
    acc_sc[...] += jnp.dot(mask, m_ref[...],
                           preferred_element_type=jnp.float32)
    sx_sc[...] += jnp.dot(mask, xs_ref[...],
                          preferred_element_type=jnp.float32)
    deg_sc[...] += jnp.sum(mask, axis=1, keepdims=True)

    @pl.when(e == pl.num_programs(1) - 1)
    def _():
        oa_ref[...] = acc_sc[...]
        os_ref[...] = sx_sc[...]
        od_ref[...] = deg_sc[...]


def _agg_partial(tgtf, msgs, xs_src, n_pad, tn, tec):
    e_loc = msgs.shape[0]
    out_ch = msgs.shape[1]
    kdim = xs_src.shape[1]
    return pl.pallas_call(
        functools.partial(_agg_body, tn=tn),
        out_shape=(jax.ShapeDtypeStruct((n_pad, out_ch), jnp.float32),
                   jax.ShapeDtypeStruct((n_pad, kdim), jnp.float32),
                   jax.ShapeDtypeStruct((n_pad, 1), jnp.float32)),
        grid=(n_pad // tn, e_loc // tec),
        in_specs=[
            pl.BlockSpec((1, tec), lambda n, e: (0, e)),
            pl.BlockSpec((tec, out_ch), lambda n, e: (e, 0)),
            pl.BlockSpec((tec, kdim), lambda n, e: (e, 0)),
        ],
        out_specs=(pl.BlockSpec((tn, out_ch), lambda n, e: (n, 0)),
                   pl.BlockSpec((tn, kdim), lambda n, e: (n, 0)),
                   pl.BlockSpec((tn, 1), lambda n, e: (n, 0))),
        scratch_shapes=[pltpu.VMEM((tn, out_ch), jnp.float32),
                        pltpu.VMEM((tn, kdim), jnp.float32),
                        pltpu.VMEM((tn, 1), jnp.float32)],
        compiler_params=_params(("parallel", "arbitrary")),
    )(tgtf, msgs, xs_src)


def _epi_body(a_ref, s_ref, d_ref, x_ref, wr_ref, b2_ref, b_ref, o_ref):
    agg = a_ref[...] + jnp.dot(s_ref[...], b2_ref[...],
                               preferred_element_type=jnp.float32)
    root = jnp.dot(x_ref[...], wr_ref[...],
                   preferred_element_type=jnp.float32)
    inv = 1.0 / jnp.maximum(d_ref[...], 1.0)
    o_ref[...] = _leaky(agg * inv + root + b_ref[...])


def _epilogue(acc, sx, deg, x_nodes, w_root, b2_mat, bias, tn):
    n_pad, out_ch = acc.shape
    kdim = x_nodes.shape[1]
    return pl.pallas_call(
        _epi_body,
        out_shape=jax.ShapeDtypeStruct((n_pad, out_ch), jnp.float32),
        grid=(n_pad // tn,),
        in_specs=[
            pl.BlockSpec((tn, out_ch), lambda n: (n, 0)),
            pl.BlockSpec((tn, kdim), lambda n: (n, 0)),
            pl.BlockSpec((tn, 1), lambda n: (n, 0)),
            pl.BlockSpec((tn, kdim), lambda n: (n, 0)),
            pl.BlockSpec((kdim, out_ch), lambda n: (0, 0)),
            pl.BlockSpec((kdim, out_ch), lambda n: (0, 0)),
            pl.BlockSpec((1, out_ch), lambda n: (0, 0)),
        ],
        out_specs=pl.BlockSpec((tn, out_ch), lambda n: (n, 0)),
        compiler_params=_params(("parallel",)),
    )(acc, sx, deg, x_nodes, w_root, b2_mat, bias)


def _head_body(bf_ref, d_ref, x_ref, w1d_ref, w1x_ref, b1_ref,
               w2_ref, b2_ref, w3_ref, b3_ref, o_ref,
               pd_sc, px_sc, cnt_sc, *, nb):
    n = pl.program_id(0)

    @pl.when(n == 0)
    def _():
        pd_sc[...] = jnp.zeros_like(pd_sc)
        px_sc[...] = jnp.zeros_like(px_sc)
        cnt_sc[...] = jnp.zeros_like(cnt_sc)

    tn = d_ref.shape[0]
    gids = jax.lax.broadcasted_iota(jnp.int32, (nb, tn), 0).astype(jnp.float32)
    mask = (gids == bf_ref[...]).astype(jnp.float32)
    pd_sc[...] += jnp.dot(mask, d_ref[...],
                          preferred_element_type=jnp.float32)
    px_sc[...] += jnp.dot(mask, x_ref[...],
                          preferred_element_type=jnp.float32)
    cnt_sc[...] += jnp.sum(mask, axis=1, keepdims=True)

    @pl.when(n == pl.num_programs(0) - 1)
    def _():
        inv = 1.0 / jnp.maximum(cnt_sc[...], 1.0)
        h = jnp.dot(pd_sc[...] * inv, w1d_ref[...],
                    preferred_element_type=jnp.float32) \
            + jnp.dot(px_sc[...] * inv, w1x_ref[...],
                      preferred_element_type=jnp.float32) + b1_ref[...]
        h = _leaky(h)
        h = _leaky(jnp.dot(h, w2_ref[...],
                           preferred_element_type=jnp.float32) + b2_ref[...])
        y = jnp.dot(h, w3_ref[...],
                    preferred_element_type=jnp.float32) + b3_ref[...]
        o_ref[...] = _sigmoid(y)


def _head(batchf, d3, x8, w1d, w1x, b1, w2, b2, w3, b3, nb, tn):
    n_pad, ddim = d3.shape
    h1 = w1d.shape[1]
    h2 = w2.shape[1]
    return pl.pallas_call(
        functools.partial(_head_body, nb=nb),
        out_shape=jax.ShapeDtypeStruct((nb, 1), jnp.float32),
        grid=(n_pad // tn,),
        in_specs=[
            pl.BlockSpec((1, tn), lambda n: (0, n)),
            pl.BlockSpec((tn, ddim), lambda n: (n, 0)),
            pl.BlockSpec((tn, 8), lambda n: (n, 0)),
            pl.BlockSpec((ddim, h1), lambda n: (0, 0)),
            pl.BlockSpec((8, h1), lambda n: (0, 0)),
            pl.BlockSpec((1, h1), lambda n: (0, 0)),
            pl.BlockSpec((h1, h2), lambda n: (0, 0)),
            pl.BlockSpec((1, h2), lambda n: (0, 0)),
            pl.BlockSpec((h2, 1), lambda n: (0, 0)),
            pl.BlockSpec((1, 1), lambda n: (0, 0)),
        ],
        out_specs=pl.BlockSpec((nb, 1), lambda n: (0, 0)),
        scratch_shapes=[pltpu.VMEM((nb, ddim), jnp.float32),
                        pltpu.VMEM((nb, 8), jnp.float32),
                        pltpu.VMEM((nb, 1), jnp.float32)],
        compiler_params=_params(("arbitrary",)),
    )(batchf, d3, x8, w1d, w1x, b1, w2, b2, w3, b3)


def kernel(x, edge_index, edge_attr, batch,
           conv1_nn_w1, conv1_nn_b1, conv1_nn_w2, conv1_nn_b2,
           conv1_root_w, conv1_bias,
           conv2_nn_w1, conv2_nn_b1, conv2_nn_w2, conv2_nn_b2,
           conv2_root_w, conv2_bias,
           conv3_nn_w1, conv3_nn_b1, conv3_nn_w2, conv3_nn_b2,
           conv3_root_w, conv3_bias,
           fc1_w, fc1_b, fc2_w, fc2_b, fc3_w, fc3_b):
    num_graphs = 64
    x = x.astype(jnp.float32)
    n_nodes, fdim = x.shape
    n_edges = edge_index.shape[1]

    e_pad = _ceil_to(n_edges, 256)
    te = 2048 if e_pad % 4096 == 0 else e_pad // 2
    tec = te
    n_pad = _ceil_to(n_nodes, 8)
    tn = 512 if n_pad % 512 == 0 else n_pad
    nb = _ceil_to(num_graphs, 8)

    srcf = jnp.full((e_pad, 1), -1.0, jnp.float32).at[:n_edges, 0].set(
        edge_index[0].astype(jnp.float32))
    tgtf = jnp.full((1, e_pad), -1.0, jnp.float32).at[0, :n_edges].set(
        edge_index[1].astype(jnp.float32))
    batchf = jnp.full((1, n_pad), -1.0, jnp.float32).at[0, :n_nodes].set(
        batch.astype(jnp.float32))

    ea8 = jnp.zeros((e_pad, 8), jnp.float32).at[:n_edges, :4].set(
        edge_attr.astype(jnp.float32))
    x_pad = jnp.zeros((n_pad, fdim), jnp.float32).at[:n_nodes].set(x)
    x8 = jnp.zeros((n_pad, 8), jnp.float32).at[:n_nodes, :4].set(x)

    devs = jax.devices()
    n_dev = 2 if len(devs) >= 2 and e_pad % (2 * te) == 0 else 1

    def fwd(ea_l, src_l, tgt_l):
        def conv(d_nodes, w1, b1, w2, b2, w_root, bias, out_ch):
            kdim = d_nodes.shape[1]
            w1p = jnp.zeros((8, w1.shape[1]), jnp.float32).at[
                :w1.shape[0]].set(w1.astype(jnp.float32))
            msgs, xs_src = _messages(ea_l, src_l, d_nodes,
                                     w2.astype(jnp.float32), w1p,
                                     b1.reshape(1, -1).astype(jnp.float32),
                                     out_ch, te)
            acc, sx, deg = _agg_partial(tgt_l, msgs, xs_src, n_pad, tn, tec)
            if n_dev > 1:
                acc, sx, deg = jax.lax.psum((acc, sx, deg), "tc")
            return _epilogue(acc, sx, deg, d_nodes,
                             w_root.astype(jnp.float32),
                             b2.astype(jnp.float32).reshape(kdim, out_ch),
                             bias.reshape(1, -1).astype(jnp.float32), tn)

        c1 = conv(x_pad, conv1_nn_w1, conv1_nn_b1, conv1_nn_w2, conv1_nn_b2,
                  conv1_root_w, conv1_bias, 256)
        d1 = jnp.concatenate([c1, x_pad], axis=1)
        c2 = conv(d1, conv2_nn_w1, conv2_nn_b1, conv2_nn_w2, conv2_nn_b2,
                  conv2_root_w, conv2_bias, 256)
        d2 = jnp.concatenate([c2, x_pad], axis=1)
        c3 = conv(d2, conv3_nn_w1, conv3_nn_b1, conv3_nn_w2, conv3_nn_b2,
                  conv3_root_w, conv3_bias, 512)

        ddim = c3.shape[1]
        w1d = fc1_w[:ddim].astype(jnp.float32)
        w1x = jnp.zeros((8, fc1_w.shape[1]), jnp.float32).at[:fdim].set(
            fc1_w[ddim:].astype(jnp.float32))
        return _head(batchf, c3, x8, w1d, w1x,
                     fc1_b.reshape(1, -1).astype(jnp.float32),
                     fc2_w.astype(jnp.float32),
                     fc2_b.reshape(1, -1).astype(jnp.float32),
                     fc3_w.astype(jnp.float32),
                     fc3_b.reshape(1, -1).astype(jnp.float32), nb, tn)

    if n_dev == 1:
        return fwd(ea8, srcf, tgtf)[:num_graphs]

    mesh = jax.sharding.Mesh(np.asarray(devs[:n_dev]), ("tc",))
    sharded = shard_map(
        fwd, mesh=mesh,
        in_specs=(jax.sharding.PartitionSpec("tc", None),
                  jax.sharding.PartitionSpec("tc", None),
                  jax.sharding.PartitionSpec(None, "tc")),
        out_specs=jax.sharding.PartitionSpec(None, None),
        check_rep=False)
    return sharded(ea8, srcf, tgtf)[:num_graphs]


# speedup vs baseline: 1.4682x; 1.0735x over previous
import functools

import jax
import jax.numpy as jnp
from jax.experimental import pallas as pl
from jax.experimental.pallas import tpu as pltpu

_SLOPE = 0.01
_KC = 4


def _ceil_to(a, b):
    return (a + b - 1) // b * b


def _leaky(v):
    return jnp.where(v >= 0, v, _SLOPE * v)


def _sigmoid(v):
    z = jnp.exp(-jnp.abs(v))
    return jnp.where(v >= 0, 1.0 / (1.0 + z), z / (1.0 + z))


def _params(dims):
    return pltpu.CompilerParams(dimension_semantics=dims,
                                vmem_limit_bytes=60 * 1024 * 1024)


def _msg_body(ea_ref, src_ref, d_ref, w1_ref, b1_ref, w2_ref,
              o_ref, oxs_ref, h_sc, xs_sc, acc_sc, *, kc, out_ch, nk, kdim):
    k = pl.program_id(1)
    te = ea_ref.shape[0]
    n_pad = d_ref.shape[0]

    @pl.when(k == 0)
    def _():
        h = jnp.dot(ea_ref[...], w1_ref[...],
                    preferred_element_type=jnp.float32) + b1_ref[...]
        h_sc[...] = _leaky(h)
        ns = 512 if n_pad % 512 == 0 else n_pad
        node = jax.lax.broadcasted_iota(jnp.int32, (te, ns), 1)
        xs = jnp.zeros((te, d_ref.shape[1]), jnp.float32)
        for i in range(n_pad // ns):
            g = (src_ref[...] == (node + i * ns).astype(jnp.float32))
            xs = xs + jnp.dot(g.astype(jnp.float32),
                              d_ref[i * ns:(i + 1) * ns, :],
                              preferred_element_type=jnp.float32)
        xs_sc[...] = xs
        oxs_ref[...] = xs
        acc_sc[...] = jnp.zeros_like(acc_sc)

    rows = jax.lax.broadcasted_iota(jnp.int32, (kdim, kc), 0)
    cols = jax.lax.broadcasted_iota(jnp.int32, (kdim, kc), 1)
    sel = (rows == k * kc + cols).astype(jnp.float32)
    xsk = jnp.dot(xs_sc[...], sel, preferred_element_type=jnp.float32)

    h = h_sc[...]
    part = jnp.dot(xsk[:, 0:1] * h, w2_ref[:, 0:out_ch],
                   preferred_element_type=jnp.float32)
    for kk in range(1, kc):
        part = part + jnp.dot(
            xsk[:, kk:kk + 1] * h, w2_ref[:, kk * out_ch:(kk + 1) * out_ch],
            preferred_element_type=jnp.float32)
    acc_sc[...] += part

    @pl.when(k == nk - 1)
    def _():
        o_ref[...] = acc_sc[...]


def _messages(ea8, srcf, d_nodes, w2, w1p, b1, out_ch, te):
    e_pad = ea8.shape[0]
    n_pad, kdim = d_nodes.shape
    hd = w1p.shape[1]
    kc = _KC
    for cand in (10, 5, 4):
        if kdim % cand == 0 and hd * cand * out_ch * 4 <= 6 * 1024 * 1024:
            kc = cand
            break
    nk = kdim // kc
    hdim = w1p.shape[1]
    return pl.pallas_call(
        functools.partial(_msg_body, kc=kc, out_ch=out_ch, nk=nk, kdim=kdim),
        out_shape=(jax.ShapeDtypeStruct((e_pad, out_ch), jnp.float32),
                   jax.ShapeDtypeStruct((e_pad, kdim), jnp.float32)),
        grid=(e_pad // te, nk),
        in_specs=[
            pl.BlockSpec((te, 8), lambda e, k: (e, 0)),
            pl.BlockSpec((te, 1), lambda e, k: (e, 0)),
            pl.BlockSpec((n_pad, kdim), lambda e, k: (0, 0)),
            pl.BlockSpec((8, hdim), lambda e, k: (0, 0)),
            pl.BlockSpec((1, hdim), lambda e, k: (0, 0)),
            pl.BlockSpec((hdim, kc * out_ch), lambda e, k: (0, k)),
        ],
        out_specs=(pl.BlockSpec((te, out_ch), lambda e, k: (e, 0)),
                   pl.BlockSpec((te, kdim), lambda e, k: (e, 0))),
        scratch_shapes=[pltpu.VMEM((te, hdim), jnp.float32),
                        pltpu.VMEM((te, kdim), jnp.float32),
                        pltpu.VMEM((te, out_ch), jnp.float32)],
        compiler_params=_params(("parallel", "arbitrary")),
    )(ea8, srcf, d_nodes, w1p, b1, w2)


def _agg_body(tgt_ref, m_ref, xs_ref, oa_ref, os_ref, od_ref,
              acc_sc, sx_sc, deg_sc, *, tn):
    n = pl.program_id(0)
    e = pl.program_id(1)

    @pl.when(e == 0)
    def _():
        acc_sc[...] = jnp.zeros_like(acc_sc)
        sx_sc[...] = jnp.zeros_like(sx_sc)
        deg_sc[...] = jnp.zeros_like(deg_sc)

    tec = m_ref.shape[0]
    rows = (jax.lax.broadcasted_iota(jnp.int32, (tn, tec), 0)
            + n * tn).astype(jnp.float32)
    mask = (rows == tgt_ref[...]).astype(jnp.float32)
    acc_sc[...] += jnp.dot(mask, m_ref[...],
                           preferred_element_type=jnp.float32)
    sx_sc[...] += jnp.dot(mask, xs_ref[...],
                          preferred_element_type=jnp.float32)
    deg_sc[...] += jnp.sum(mask, axis=1, keepdims=True)

    @pl.when(e == pl.num_programs(1) - 1)
    def _():
        oa_ref[...] = acc_sc[...]
        os_ref[...] = sx_sc[...]
        od_ref[...] = deg_sc[...]


def _agg_partial(tgtf, msgs, xs_src, n_pad, tn, tec):
    e_loc = msgs.shape[0]
    out_ch = msgs.shape[1]
    kdim = xs_src.shape[1]
    return pl.pallas_call(
        functools.partial(_agg_body, tn=tn),
        out_shape=(jax.ShapeDtypeStruct((n_pad, out_ch), jnp.float32),
                   jax.ShapeDtypeStruct((n_pad, kdim), jnp.float32),
                   jax.ShapeDtypeStruct((n_pad, 1), jnp.float32)),
        grid=(n_pad // tn, e_loc // tec),
        in_specs=[
            pl.BlockSpec((1, tec), lambda n, e: (0, e)),
            pl.BlockSpec((tec, out_ch), lambda n, e: (e, 0)),
            pl.BlockSpec((tec, kdim), lambda n, e: (e, 0)),
        ],
        out_specs=(pl.BlockSpec((tn, out_ch), lambda n, e: (n, 0)),
                   pl.BlockSpec((tn, kdim), lambda n, e: (n, 0)),
                   pl.BlockSpec((tn, 1), lambda n, e: (n, 0))),
        scratch_shapes=[pltpu.VMEM((tn, out_ch), jnp.float32),
                        pltpu.VMEM((tn, kdim), jnp.float32),
                        pltpu.VMEM((tn, 1), jnp.float32)],
        compiler_params=_params(("parallel", "arbitrary")),
    )(tgtf, msgs, xs_src)


def _epi_body(a_ref, s_ref, d_ref, x_ref, wr_ref, b2_ref, b_ref, o_ref):
    agg = a_ref[...] + jnp.dot(s_ref[...], b2_ref[...],
                               preferred_element_type=jnp.float32)
    root = jnp.dot(x_ref[...], wr_ref[...],
                   preferred_element_type=jnp.float32)
    inv = 1.0 / jnp.maximum(d_ref[...], 1.0)
    o_ref[...] = _leaky(agg * inv + root + b_ref[...])


def _epilogue(acc, sx, deg, x_nodes, w_root, b2_mat, bias, tn):
    n_pad, out_ch = acc.shape
    kdim = x_nodes.shape[1]
    return pl.pallas_call(
        _epi_body,
        out_shape=jax.ShapeDtypeStruct((n_pad, out_ch), jnp.float32),
        grid=(n_pad // tn,),
        in_specs=[
            pl.BlockSpec((tn, out_ch), lambda n: (n, 0)),
            pl.BlockSpec((tn, kdim), lambda n: (n, 0)),
            pl.BlockSpec((tn, 1), lambda n: (n, 0)),
            pl.BlockSpec((tn, kdim), lambda n: (n, 0)),
            pl.BlockSpec((kdim, out_ch), lambda n: (0, 0)),
            pl.BlockSpec((kdim, out_ch), lambda n: (0, 0)),
            pl.BlockSpec((1, out_ch), lambda n: (0, 0)),
        ],
        out_specs=pl.BlockSpec((tn, out_ch), lambda n: (n, 0)),
        compiler_params=_params(("parallel",)),
    )(acc, sx, deg, x_nodes, w_root, b2_mat, bias)


def _head_body(bf_ref, d_ref, x_ref, w1d_ref, w1x_ref, b1_ref,
               w2_ref, b2_ref, w3_ref, b3_ref, o_ref,
               pd_sc, px_sc, cnt_sc, *, nb):
    n = pl.program_id(0)

    @pl.when(n == 0)
    def _():
        pd_sc[...] = jnp.zeros_like(pd_sc)
        px_sc[...] = jnp.zeros_like(px_sc)
        cnt_sc[...] = jnp.zeros_like(cnt_sc)

    tn = d_ref.shape[0]
    gids = jax.lax.broadcasted_iota(jnp.int32, (nb, tn), 0).astype(jnp.float32)
    mask = (gids == bf_ref[...]).astype(jnp.float32)
    pd_sc[...] += jnp.dot(mask, d_ref[...],
                          preferred_element_type=jnp.float32)
    px_sc[...] += jnp.dot(mask, x_ref[...],
                          preferred_element_type=jnp.float32)
    cnt_sc[...] += jnp.sum(mask, axis=1, keepdims=True)

    @pl.when(n == pl.num_programs(0) - 1)
    def _():
        inv = 1.0 / jnp.maximum(cnt_sc[...], 1.0)
        h = jnp.dot(pd_sc[...] * inv, w1d_ref[...],
                    preferred_element_type=jnp.float32) \
            + jnp.dot(px_sc[...] * inv, w1x_ref[...],
                      preferred_element_type=jnp.float32) + b1_ref[...]
        h = _leaky(h)
        h = _leaky(jnp.dot(h, w2_ref[...],
                           preferred_element_type=jnp.float32) + b2_ref[...])
        y = jnp.dot(h, w3_ref[...],
                    preferred_element_type=jnp.float32) + b3_ref[...]
        o_ref[...] = _sigmoid(y)


def _head(batchf, d3, x8, w1d, w1x, b1, w2, b2, w3, b3, nb, tn):
    n_pad, ddim = d3.shape
    h1 = w1d.shape[1]
    h2 = w2.shape[1]
    return pl.pallas_call(
        functools.partial(_head_body, nb=nb),
        out_shape=jax.ShapeDtypeStruct((nb, 1), jnp.float32),
        grid=(n_pad // tn,),
        in_specs=[
            pl.BlockSpec((1, tn), lambda n: (0, n)),
            pl.BlockSpec((tn, ddim), lambda n: (n, 0)),
            pl.BlockSpec((tn, 8), lambda n: (n, 0)),
            pl.BlockSpec((ddim, h1), lambda n: (0, 0)),
            pl.BlockSpec((8, h1), lambda n: (0, 0)),
            pl.BlockSpec((1, h1), lambda n: (0, 0)),
            pl.BlockSpec((h1, h2), lambda n: (0, 0)),
            pl.BlockSpec((1, h2), lambda n: (0, 0)),
            pl.BlockSpec((h2, 1), lambda n: (0, 0)),
            pl.BlockSpec((1, 1), lambda n: (0, 0)),
        ],
        out_specs=pl.BlockSpec((nb, 1), lambda n: (0, 0)),
        scratch_shapes=[pltpu.VMEM((nb, ddim), jnp.float32),
                        pltpu.VMEM((nb, 8), jnp.float32),
                        pltpu.VMEM((nb, 1), jnp.float32)],
        compiler_params=_params(("arbitrary",)),
    )(batchf, d3, x8, w1d, w1x, b1, w2, b2, w3, b3)


def kernel(x, edge_index, edge_attr, batch,
           conv1_nn_w1, conv1_nn_b1, conv1_nn_w2, conv1_nn_b2,
           conv1_root_w, conv1_bias,
           conv2_nn_w1, conv2_nn_b1, conv2_nn_w2, conv2_nn_b2,
           conv2_root_w, conv2_bias,
           conv3_nn_w1, conv3_nn_b1, conv3_nn_w2, conv3_nn_b2,
           conv3_root_w, conv3_bias,
           fc1_w, fc1_b, fc2_w, fc2_b, fc3_w, fc3_b):
    num_graphs = 64
    x = x.astype(jnp.float32)
    n_nodes, fdim = x.shape
    n_edges = edge_index.shape[1]

    e_pad = _ceil_to(n_edges, 256)
    te = 2048 if e_pad % 4096 == 0 else e_pad // 2
    tec = te
    n_pad = _ceil_to(n_nodes, 8)
    tn = 512 if n_pad % 512 == 0 else n_pad
    nb = _ceil_to(num_graphs, 8)

    srcf = jnp.full((e_pad, 1), -1.0, jnp.float32).at[:n_edges, 0].set(
        edge_index[0].astype(jnp.float32))
    tgtf = jnp.full((1, e_pad), -1.0, jnp.float32).at[0, :n_edges].set(
        edge_index[1].astype(jnp.float32))
    batchf = jnp.full((1, n_pad), -1.0, jnp.float32).at[0, :n_nodes].set(
        batch.astype(jnp.float32))

    ea8 = jnp.zeros((e_pad, 8), jnp.float32).at[:n_edges, :4].set(
        edge_attr.astype(jnp.float32))
    x_pad = jnp.zeros((n_pad, fdim), jnp.float32).at[:n_nodes].set(x)
    x8 = jnp.zeros((n_pad, 8), jnp.float32).at[:n_nodes, :4].set(x)

    def fwd(ea_l, src_l, tgt_l):
        def conv(d_nodes, w1, b1, w2, b2, w_root, bias, out_ch):
            kdim = d_nodes.shape[1]
            w1p = jnp.zeros((8, w1.shape[1]), jnp.float32).at[
                :w1.shape[0]].set(w1.astype(jnp.float32))
            msgs, xs_src = _messages(ea_l, src_l, d_nodes,
                                     w2.astype(jnp.float32), w1p,
                                     b1.reshape(1, -1).astype(jnp.float32),
                                     out_ch, te)
            acc, sx, deg = _agg_partial(tgt_l, msgs, xs_src, n_pad, tn, tec)
            return _epilogue(acc, sx, deg, d_nodes,
                             w_root.astype(jnp.float32),
                             b2.astype(jnp.float32).reshape(kdim, out_ch),
                             bias.reshape(1, -1).astype(jnp.float32), tn)

        c1 = conv(x_pad, conv1_nn_w1, conv1_nn_b1, conv1_nn_w2, conv1_nn_b2,
                  conv1_root_w, conv1_bias, 256)
        d1 = jnp.concatenate([c1, x_pad], axis=1)
        c2 = conv(d1, conv2_nn_w1, conv2_nn_b1, conv2_nn_w2, conv2_nn_b2,
                  conv2_root_w, conv2_bias, 256)
        d2 = jnp.concatenate([c2, x_pad], axis=1)
        c3 = conv(d2, conv3_nn_w1, conv3_nn_b1, conv3_nn_w2, conv3_nn_b2,
                  conv3_root_w, conv3_bias, 512)

        ddim = c3.shape[1]
        w1d = fc1_w[:ddim].astype(jnp.float32)
        w1x = jnp.zeros((8, fc1_w.shape[1]), jnp.float32).at[:fdim].set(
            fc1_w[ddim:].astype(jnp.float32))
        return _head(batchf, c3, x8, w1d, w1x,
                     fc1_b.reshape(1, -1).astype(jnp.float32),
                     fc2_w.astype(jnp.float32),
                     fc2_b.reshape(1, -1).astype(jnp.float32),
                     fc3_w.astype(jnp.float32),
                     fc3_b.reshape(1, -1).astype(jnp.float32), nb, tn)

    return fwd(ea8, srcf, tgtf)[:num_graphs]


# speedup vs baseline: 1.4796x; 1.0078x over previous
import functools

import jax
import jax.numpy as jnp
from jax.experimental import pallas as pl
from jax.experimental.pallas import tpu as pltpu

_SLOPE = 0.01
_KC = 4


def _ceil_to(a, b):
    return (a + b - 1) // b * b


def _leaky(v):
    return jnp.where(v >= 0, v, _SLOPE * v)


def _sigmoid(v):
    z = jnp.exp(-jnp.abs(v))
    return jnp.where(v >= 0, 1.0 / (1.0 + z), z / (1.0 + z))


def _params(dims):
    return pltpu.CompilerParams(dimension_semantics=dims,
                                vmem_limit_bytes=60 * 1024 * 1024)


def _msg_body(ea_ref, src_ref, d_ref, w1_ref, b1_ref, w2_ref,
              o_ref, oxs_ref, h_sc, xs_sc, acc_sc, *, kc, out_ch, nk, kdim):
    k = pl.program_id(1)
    te = ea_ref.shape[0]
    n_pad = d_ref.shape[0]

    @pl.when(k == 0)
    def _():
        h = jnp.dot(ea_ref[...], w1_ref[...],
                    preferred_element_type=jnp.float32) + b1_ref[...]
        h_sc[...] = _leaky(h)
        ns = 512 if n_pad % 512 == 0 else n_pad
        node = jax.lax.broadcasted_iota(jnp.int32, (te, ns), 1)
        xs = jnp.zeros((te, d_ref.shape[1]), jnp.float32)
        for i in range(n_pad // ns):
            g = (src_ref[...] == (node + i * ns).astype(jnp.float32))
            xs = xs + jnp.dot(g.astype(jnp.float32),
                              d_ref[i * ns:(i + 1) * ns, :],
                              preferred_element_type=jnp.float32)
        xs_sc[...] = xs
        oxs_ref[...] = xs
        acc_sc[...] = jnp.zeros_like(acc_sc)

    rows = jax.lax.broadcasted_iota(jnp.int32, (kdim, kc), 0)
    cols = jax.lax.broadcasted_iota(jnp.int32, (kdim, kc), 1)
    sel = (rows == k * kc + cols).astype(jnp.float32)
    xsk = jnp.dot(xs_sc[...], sel, preferred_element_type=jnp.float32)

    h = h_sc[...]
    part = jnp.dot(xsk[:, 0:1] * h, w2_ref[:, 0:out_ch],
                   preferred_element_type=jnp.float32)
    for kk in range(1, kc):
        part = part + jnp.dot(
            xsk[:, kk:kk + 1] * h, w2_ref[:, kk * out_ch:(kk + 1) * out_ch],
            preferred_element_type=jnp.float32)
    acc_sc[...] += part

    @pl.when(k == nk - 1)
    def _():
        o_ref[...] = acc_sc[...]


def _messages(ea8, srcf, d_nodes, w2, w1p, b1, out_ch, te):
    e_pad = ea8.shape[0]
    n_pad, kdim = d_nodes.shape
    hd = w1p.shape[1]
    kc = _KC
    for cand in (10, 5, 4):
        if kdim % cand == 0 and hd * cand * out_ch * 4 <= 6 * 1024 * 1024:
            kc = cand
            break
    nk = kdim // kc
    hdim = w1p.shape[1]
    return pl.pallas_call(
        functools.partial(_msg_body, kc=kc, out_ch=out_ch, nk=nk, kdim=kdim),
        out_shape=(jax.ShapeDtypeStruct((e_pad, out_ch), jnp.float32),
                   jax.ShapeDtypeStruct((e_pad, kdim), jnp.float32)),
        grid=(e_pad // te, nk),
        in_specs=[
            pl.BlockSpec((te, 8), lambda e, k: (e, 0)),
            pl.BlockSpec((te, 1), lambda e, k: (e, 0)),
            pl.BlockSpec((n_pad, kdim), lambda e, k: (0, 0)),
            pl.BlockSpec((8, hdim), lambda e, k: (0, 0)),
            pl.BlockSpec((1, hdim), lambda e, k: (0, 0)),
            pl.BlockSpec((hdim, kc * out_ch), lambda e, k: (0, k)),
        ],
        out_specs=(pl.BlockSpec((te, out_ch), lambda e, k: (e, 0)),
                   pl.BlockSpec((te, kdim), lambda e, k: (e, 0))),
        scratch_shapes=[pltpu.VMEM((te, hdim), jnp.float32),
                        pltpu.VMEM((te, kdim), jnp.float32),
                        pltpu.VMEM((te, out_ch), jnp.float32)],
        compiler_params=_params(("parallel", "arbitrary")),
    )(ea8, srcf, d_nodes, w1p, b1, w2)


def _agg_body(tgt_ref, m_ref, xs_ref, x_ref, wr_ref, b2_ref, b_ref, o_ref,
              acc_sc, sx_sc, deg_sc, *, tn):
    n = pl.program_id(0)
    e = pl.program_id(1)

    @pl.when(e == 0)
    def _():
        acc_sc[...] = jnp.zeros_like(acc_sc)
        sx_sc[...] = jnp.zeros_like(sx_sc)
        deg_sc[...] = jnp.zeros_like(deg_sc)

    tec = m_ref.shape[0]
    rows = (jax.lax.broadcasted_iota(jnp.int32, (tn, tec), 0)
            + n * tn).astype(jnp.float32)
    mask = (rows == tgt_ref[...]).astype(jnp.float32)
    acc_sc[...] += jnp.dot(mask, m_ref[...],
                           preferred_element_type=jnp.float32)
    sx_sc[...] += jnp.dot(mask, xs_ref[...],
                          preferred_element_type=jnp.float32)
    deg_sc[...] += jnp.sum(mask, axis=1, keepdims=True)

    @pl.when(e == pl.num_programs(1) - 1)
    def _():
        agg = acc_sc[...] + jnp.dot(sx_sc[...], b2_ref[...],
                                    preferred_element_type=jnp.float32)
        root = jnp.dot(x_ref[...], wr_ref[...],
                       preferred_element_type=jnp.float32)
        inv = 1.0 / jnp.maximum(deg_sc[...], 1.0)
        o_ref[...] = _leaky(agg * inv + root + b_ref[...])


def _combine(tgtf, msgs, xs_src, x_nodes, w_root, b2_mat, bias, tn, tec):
    n_pad = x_nodes.shape[0]
    e_pad = msgs.shape[0]
    out_ch = msgs.shape[1]
    kdim = xs_src.shape[1]
    return pl.pallas_call(
        functools.partial(_agg_body, tn=tn),
        out_shape=jax.ShapeDtypeStruct((n_pad, out_ch), jnp.float32),
        grid=(n_pad // tn, e_pad // tec),
        in_specs=[
            pl.BlockSpec((1, tec), lambda n, e: (0, e)),
            pl.BlockSpec((tec, out_ch), lambda n, e: (e, 0)),
            pl.BlockSpec((tec, kdim), lambda n, e: (e, 0)),
            pl.BlockSpec((tn, kdim), lambda n, e: (n, 0)),
            pl.BlockSpec((kdim, out_ch), lambda n, e: (0, 0)),
            pl.BlockSpec((kdim, out_ch), lambda n, e: (0, 0)),
            pl.BlockSpec((1, out_ch), lambda n, e: (0, 0)),
        ],
        out_specs=pl.BlockSpec((tn, out_ch), lambda n, e: (n, 0)),
        scratch_shapes=[pltpu.VMEM((tn, out_ch), jnp.float32),
                        pltpu.VMEM((tn, kdim), jnp.float32),
                        pltpu.VMEM((tn, 1), jnp.float32)],
        compiler_params=_params(("parallel", "arbitrary")),
    )(tgtf, msgs, xs_src, x_nodes, w_root, b2_mat, bias)


def _head_body(bf_ref, d_ref, x_ref, w1d_ref, w1x_ref, b1_ref,
               w2_ref, b2_ref, w3_ref, b3_ref, o_ref,
               pd_sc, px_sc, cnt_sc, *, nb):
    n = pl.program_id(0)

    @pl.when(n == 0)
    def _():
        pd_sc[...] = jnp.zeros_like(pd_sc)
        px_sc[...] = jnp.zeros_like(px_sc)
        cnt_sc[...] = jnp.zeros_like(cnt_sc)

    tn = d_ref.shape[0]
    gids = jax.lax.broadcasted_iota(jnp.int32, (nb, tn), 0).astype(jnp.float32)
    mask = (gids == bf_ref[...]).astype(jnp.float32)
    pd_sc[...] += jnp.dot(mask, d_ref[...],
                          preferred_element_type=jnp.float32)
    px_sc[...] += jnp.dot(mask, x_ref[...],
                          preferred_element_type=jnp.float32)
    cnt_sc[...] += jnp.sum(mask, axis=1, keepdims=True)

    @pl.when(n == pl.num_programs(0) - 1)
    def _():
        inv = 1.0 / jnp.maximum(cnt_sc[...], 1.0)
        h = jnp.dot(pd_sc[...] * inv, w1d_ref[...],
                    preferred_element_type=jnp.float32) \
            + jnp.dot(px_sc[...] * inv, w1x_ref[...],
                      preferred_element_type=jnp.float32) + b1_ref[...]
        h = _leaky(h)
        h = _leaky(jnp.dot(h, w2_ref[...],
                           preferred_element_type=jnp.float32) + b2_ref[...])
        y = jnp.dot(h, w3_ref[...],
                    preferred_element_type=jnp.float32) + b3_ref[...]
        o_ref[...] = _sigmoid(y)


def _head(batchf, d3, x8, w1d, w1x, b1, w2, b2, w3, b3, nb, tn):
    n_pad, ddim = d3.shape
    h1 = w1d.shape[1]
    h2 = w2.shape[1]
    return pl.pallas_call(
        functools.partial(_head_body, nb=nb),
        out_shape=jax.ShapeDtypeStruct((nb, 1), jnp.float32),
        grid=(n_pad // tn,),
        in_specs=[
            pl.BlockSpec((1, tn), lambda n: (0, n)),
            pl.BlockSpec((tn, ddim), lambda n: (n, 0)),
            pl.BlockSpec((tn, 8), lambda n: (n, 0)),
            pl.BlockSpec((ddim, h1), lambda n: (0, 0)),
            pl.BlockSpec((8, h1), lambda n: (0, 0)),
            pl.BlockSpec((1, h1), lambda n: (0, 0)),
            pl.BlockSpec((h1, h2), lambda n: (0, 0)),
            pl.BlockSpec((1, h2), lambda n: (0, 0)),
            pl.BlockSpec((h2, 1), lambda n: (0, 0)),
            pl.BlockSpec((1, 1), lambda n: (0, 0)),
        ],
        out_specs=pl.BlockSpec((nb, 1), lambda n: (0, 0)),
        scratch_shapes=[pltpu.VMEM((nb, ddim), jnp.float32),
                        pltpu.VMEM((nb, 8), jnp.float32),
                        pltpu.VMEM((nb, 1), jnp.float32)],
        compiler_params=_params(("arbitrary",)),
    )(batchf, d3, x8, w1d, w1x, b1, w2, b2, w3, b3)


def kernel(x, edge_index, edge_attr, batch,
           conv1_nn_w1, conv1_nn_b1, conv1_nn_w2, conv1_nn_b2,
           conv1_root_w, conv1_bias,
           conv2_nn_w1, conv2_nn_b1, conv2_nn_w2, conv2_nn_b2,
           conv2_root_w, conv2_bias,
           conv3_nn_w1, conv3_nn_b1, conv3_nn_w2, conv3_nn_b2,
           conv3_root_w, conv3_bias,
           fc1_w, fc1_b, fc2_w, fc2_b, fc3_w, fc3_b):
    num_graphs = 64
    x = x.astype(jnp.float32)
    n_nodes, fdim = x.shape
    n_edges = edge_index.shape[1]

    e_pad = _ceil_to(n_edges, 256)
    te = 2048 if e_pad % 4096 == 0 else e_pad // 2
    tec = te
    n_pad = _ceil_to(n_nodes, 8)
    tn = 512 if n_pad % 512 == 0 else n_pad
    nb = _ceil_to(num_graphs, 8)

    srcf = jnp.full((e_pad, 1), -1.0, jnp.float32).at[:n_edges, 0].set(
        edge_index[0].astype(jnp.float32))
    tgtf = jnp.full((1, e_pad), -1.0, jnp.float32).at[0, :n_edges].set(
        edge_index[1].astype(jnp.float32))
    batchf = jnp.full((1, n_pad), -1.0, jnp.float32).at[0, :n_nodes].set(
        batch.astype(jnp.float32))

    ea8 = jnp.zeros((e_pad, 8), jnp.float32).at[:n_edges, :4].set(
        edge_attr.astype(jnp.float32))
    x_pad = jnp.zeros((n_pad, fdim), jnp.float32).at[:n_nodes].set(x)
    x8 = jnp.zeros((n_pad, 8), jnp.float32).at[:n_nodes, :4].set(x)

    def fwd(ea_l, src_l, tgt_l):
        def conv(d_nodes, w1, b1, w2, b2, w_root, bias, out_ch):
            kdim = d_nodes.shape[1]
            w1p = jnp.zeros((8, w1.shape[1]), jnp.float32).at[
                :w1.shape[0]].set(w1.astype(jnp.float32))
            msgs, xs_src = _messages(ea_l, src_l, d_nodes,
                                     w2.astype(jnp.float32), w1p,
                                     b1.reshape(1, -1).astype(jnp.float32),
                                     out_ch, te)
            return _combine(tgt_l, msgs, xs_src, d_nodes,
                            w_root.astype(jnp.float32),
                            b2.astype(jnp.float32).reshape(kdim, out_ch),
                            bias.reshape(1, -1).astype(jnp.float32), tn, tec)

        c1 = conv(x_pad, conv1_nn_w1, conv1_nn_b1, conv1_nn_w2, conv1_nn_b2,
                  conv1_root_w, conv1_bias, 256)
        d1 = jnp.concatenate([c1, x_pad], axis=1)
        c2 = conv(d1, conv2_nn_w1, conv2_nn_b1, conv2_nn_w2, conv2_nn_b2,
                  conv2_root_w, conv2_bias, 256)
        d2 = jnp.concatenate([c2, x_pad], axis=1)
        c3 = conv(d2, conv3_nn_w1, conv3_nn_b1, conv3_nn_w2, conv3_nn_b2,
                  conv3_root_w, conv3_bias, 512)

        ddim = c3.shape[1]
        w1d = fc1_w[:ddim].astype(jnp.float32)
        w1x = jnp.zeros((8, fc1_w.shape[1]), jnp.float32).at[:fdim].set(
            fc1_w[ddim:].astype(jnp.float32))
        return _head(batchf, c3, x8, w1d, w1x,
                     fc1_b.reshape(1, -1).astype(jnp.float32),
                     fc2_w.astype(jnp.float32),
                     fc2_b.reshape(1, -1).astype(jnp.float32),
                     fc3_w.astype(jnp.float32),
                     fc3_b.reshape(1, -1).astype(jnp.float32), nb, tn)

    return fwd(ea8, srcf, tgtf)[:num_graphs]


# speedup vs baseline: 1.5059x; 1.0178x over previous
import functools

import jax
import jax.numpy as jnp
from jax.experimental import pallas as pl
from jax.experimental.pallas import tpu as pltpu

_SLOPE = 0.01
_KC = 4


def _ceil_to(a, b):
    return (a + b - 1) // b * b


def _leaky(v):
    return jnp.where(v >= 0, v, _SLOPE * v)


def _sigmoid(v):
    z = jnp.exp(-jnp.abs(v))
    return jnp.where(v >= 0, 1.0 / (1.0 + z), z / (1.0 + z))


def _params(dims):
    return pltpu.CompilerParams(dimension_semantics=dims,
                                vmem_limit_bytes=60 * 1024 * 1024)


def _msg_body(ea_ref, src_ref, d_ref, w1_ref, b1_ref, w2_ref,
              o_ref, oxs_ref, h_sc, xs_sc, acc_sc, *, kc, out_ch, nk, kdim):
    k = pl.program_id(1)
    te = ea_ref.shape[0]
    n_pad = d_ref.shape[0]

    @pl.when(k == 0)
    def _():
        h = jnp.dot(ea_ref[...], w1_ref[...],
                    preferred_element_type=jnp.float32) + b1_ref[...]
        h_sc[...] = _leaky(h)
        ns = 512 if n_pad % 512 == 0 else n_pad
        node = jax.lax.broadcasted_iota(jnp.int32, (te, ns), 1)
        xs = jnp.zeros((te, d_ref.shape[1]), jnp.float32)
        for i in range(n_pad // ns):
            g = (src_ref[...] == (node + i * ns).astype(jnp.float32))
            xs = xs + jnp.dot(g.astype(jnp.float32),
                              d_ref[i * ns:(i + 1) * ns, :],
                              preferred_element_type=jnp.float32)
        xs_sc[...] = xs
        oxs_ref[...] = xs
        acc_sc[...] = jnp.zeros_like(acc_sc)

    rows = jax.lax.broadcasted_iota(jnp.int32, (kdim, kc), 0)
    cols = jax.lax.broadcasted_iota(jnp.int32, (kdim, kc), 1)
    sel = (rows == k * kc + cols).astype(jnp.float32)
    xsk = jnp.dot(xs_sc[...], sel, preferred_element_type=jnp.float32)

    h = h_sc[...]
    part = jnp.dot(xsk[:, 0:1] * h, w2_ref[:, 0:out_ch],
                   preferred_element_type=jnp.float32)
    for kk in range(1, kc):
        part = part + jnp.dot(
            xsk[:, kk:kk + 1] * h, w2_ref[:, kk * out_ch:(kk + 1) * out_ch],
            preferred_element_type=jnp.float32)
    acc_sc[...] += part

    @pl.when(k == nk - 1)
    def _():
        o_ref[...] = acc_sc[...]


def _messages(ea8, srcf, d_nodes, w2, w1p, b1, out_ch, te):
    e_pad = ea8.shape[0]
    n_pad, kdim = d_nodes.shape
    hd = w1p.shape[1]
    kc = _KC
    for cand in (10, 5, 4):
        if kdim % cand == 0:
            kc = cand
            break
    nk = kdim // kc
    if hd * kc * out_ch * 4 > 6 * 1024 * 1024 and te % 256 == 0:
        te = te // 2
    hdim = w1p.shape[1]
    return pl.pallas_call(
        functools.partial(_msg_body, kc=kc, out_ch=out_ch, nk=nk, kdim=kdim),
        out_shape=(jax.ShapeDtypeStruct((e_pad, out_ch), jnp.float32),
                   jax.ShapeDtypeStruct((e_pad, kdim), jnp.float32)),
        grid=(e_pad // te, nk),
        in_specs=[
            pl.BlockSpec((te, 8), lambda e, k: (e, 0)),
            pl.BlockSpec((te, 1), lambda e, k: (e, 0)),
            pl.BlockSpec((n_pad, kdim), lambda e, k: (0, 0)),
            pl.BlockSpec((8, hdim), lambda e, k: (0, 0)),
            pl.BlockSpec((1, hdim), lambda e, k: (0, 0)),
            pl.BlockSpec((hdim, kc * out_ch), lambda e, k: (0, k)),
        ],
        out_specs=(pl.BlockSpec((te, out_ch), lambda e, k: (e, 0)),
                   pl.BlockSpec((te, kdim), lambda e, k: (e, 0))),
        scratch_shapes=[pltpu.VMEM((te, hdim), jnp.float32),
                        pltpu.VMEM((te, kdim), jnp.float32),
                        pltpu.VMEM((te, out_ch), jnp.float32)],
        compiler_params=_params(("parallel", "arbitrary")),
    )(ea8, srcf, d_nodes, w1p, b1, w2)


def _agg_body(tgt_ref, m_ref, xs_ref, x_ref, wr_ref, b2_ref, b_ref, o_ref,
              acc_sc, sx_sc, deg_sc, *, tn):
    n = pl.program_id(0)
    e = pl.program_id(1)

    @pl.when(e == 0)
    def _():
        acc_sc[...] = jnp.zeros_like(acc_sc)
        sx_sc[...] = jnp.zeros_like(sx_sc)
        deg_sc[...] = jnp.zeros_like(deg_sc)

    tec = m_ref.shape[0]
    rows = (jax.lax.broadcasted_iota(jnp.int32, (tn, tec), 0)
            + n * tn).astype(jnp.float32)
    mask = (rows == tgt_ref[...]).astype(jnp.float32)
    acc_sc[...] += jnp.dot(mask, m_ref[...],
                           preferred_element_type=jnp.float32)
    sx_sc[...] += jnp.dot(mask, xs_ref[...],
                          preferred_element_type=jnp.float32)
    deg_sc[...] += jnp.sum(mask, axis=1, keepdims=True)

    @pl.when(e == pl.num_programs(1) - 1)
    def _():
        agg = acc_sc[...] + jnp.dot(sx_sc[...], b2_ref[...],
                                    preferred_element_type=jnp.float32)
        root = jnp.dot(x_ref[...], wr_ref[...],
                       preferred_element_type=jnp.float32)
        inv = 1.0 / jnp.maximum(deg_sc[...], 1.0)
        o_ref[...] = _leaky(agg * inv + root + b_ref[...])


def _combine(tgtf, msgs, xs_src, x_nodes, w_root, b2_mat, bias, tn, tec):
    n_pad = x_nodes.shape[0]
    e_pad = msgs.shape[0]
    out_ch = msgs.shape[1]
    kdim = xs_src.shape[1]
    return pl.pallas_call(
        functools.partial(_agg_body, tn=tn),
        out_shape=jax.ShapeDtypeStruct((n_pad, out_ch), jnp.float32),
        grid=(n_pad // tn, e_pad // tec),
        in_specs=[
            pl.BlockSpec((1, tec), lambda n, e: (0, e)),
            pl.BlockSpec((tec, out_ch), lambda n, e: (e, 0)),
            pl.BlockSpec((tec, kdim), lambda n, e: (e, 0)),
            pl.BlockSpec((tn, kdim), lambda n, e: (n, 0)),
            pl.BlockSpec((kdim, out_ch), lambda n, e: (0, 0)),
            pl.BlockSpec((kdim, out_ch), lambda n, e: (0, 0)),
            pl.BlockSpec((1, out_ch), lambda n, e: (0, 0)),
        ],
        out_specs=pl.BlockSpec((tn, out_ch), lambda n, e: (n, 0)),
        scratch_shapes=[pltpu.VMEM((tn, out_ch), jnp.float32),
                        pltpu.VMEM((tn, kdim), jnp.float32),
                        pltpu.VMEM((tn, 1), jnp.float32)],
        compiler_params=_params(("parallel", "arbitrary")),
    )(tgtf, msgs, xs_src, x_nodes, w_root, b2_mat, bias)


def _head_body(bf_ref, d_ref, x_ref, w1d_ref, w1x_ref, b1_ref,
               w2_ref, b2_ref, w3_ref, b3_ref, o_ref,
               pd_sc, px_sc, cnt_sc, *, nb):
    n = pl.program_id(0)

    @pl.when(n == 0)
    def _():
        pd_sc[...] = jnp.zeros_like(pd_sc)
        px_sc[...] = jnp.zeros_like(px_sc)
        cnt_sc[...] = jnp.zeros_like(cnt_sc)

    tn = d_ref.shape[0]
    gids = jax.lax.broadcasted_iota(jnp.int32, (nb, tn), 0).astype(jnp.float32)
    mask = (gids == bf_ref[...]).astype(jnp.float32)
    pd_sc[...] += jnp.dot(mask, d_ref[...],
                          preferred_element_type=jnp.float32)
    px_sc[...] += jnp.dot(mask, x_ref[...],
                          preferred_element_type=jnp.float32)
    cnt_sc[...] += jnp.sum(mask, axis=1, keepdims=True)

    @pl.when(n == pl.num_programs(0) - 1)
    def _():
        inv = 1.0 / jnp.maximum(cnt_sc[...], 1.0)
        h = jnp.dot(pd_sc[...] * inv, w1d_ref[...],
                    preferred_element_type=jnp.float32) \
            + jnp.dot(px_sc[...] * inv, w1x_ref[...],
                      preferred_element_type=jnp.float32) + b1_ref[...]
        h = _leaky(h)
        h = _leaky(jnp.dot(h, w2_ref[...],
                           preferred_element_type=jnp.float32) + b2_ref[...])
        y = jnp.dot(h, w3_ref[...],
                    preferred_element_type=jnp.float32) + b3_ref[...]
        o_ref[...] = _sigmoid(y)


def _head(batchf, d3, x8, w1d, w1x, b1, w2, b2, w3, b3, nb, tn):
    n_pad, ddim = d3.shape
    h1 = w1d.shape[1]
    h2 = w2.shape[1]
    return pl.pallas_call(
        functools.partial(_head_body, nb=nb),
        out_shape=jax.ShapeDtypeStruct((nb, 1), jnp.float32),
        grid=(n_pad // tn,),
        in_specs=[
            pl.BlockSpec((1, tn), lambda n: (0, n)),
            pl.BlockSpec((tn, ddim), lambda n: (n, 0)),
            pl.BlockSpec((tn, 8), lambda n: (n, 0)),
            pl.BlockSpec((ddim, h1), lambda n: (0, 0)),
            pl.BlockSpec((8, h1), lambda n: (0, 0)),
            pl.BlockSpec((1, h1), lambda n: (0, 0)),
            pl.BlockSpec((h1, h2), lambda n: (0, 0)),
            pl.BlockSpec((1, h2), lambda n: (0, 0)),
            pl.BlockSpec((h2, 1), lambda n: (0, 0)),
            pl.BlockSpec((1, 1), lambda n: (0, 0)),
        ],
        out_specs=pl.BlockSpec((nb, 1), lambda n: (0, 0)),
        scratch_shapes=[pltpu.VMEM((nb, ddim), jnp.float32),
                        pltpu.VMEM((nb, 8), jnp.float32),
                        pltpu.VMEM((nb, 1), jnp.float32)],
        compiler_params=_params(("arbitrary",)),
    )(batchf, d3, x8, w1d, w1x, b1, w2, b2, w3, b3)


def kernel(x, edge_index, edge_attr, batch,
           conv1_nn_w1, conv1_nn_b1, conv1_nn_w2, conv1_nn_b2,
           conv1_root_w, conv1_bias,
           conv2_nn_w1, conv2_nn_b1, conv2_nn_w2, conv2_nn_b2,
           conv2_root_w, conv2_bias,
           conv3_nn_w1, conv3_nn_b1, conv3_nn_w2, conv3_nn_b2,
           conv3_root_w, conv3_bias,
           fc1_w, fc1_b, fc2_w, fc2_b, fc3_w, fc3_b):
    num_graphs = 64
    x = x.astype(jnp.float32)
    n_nodes, fdim = x.shape
    n_edges = edge_index.shape[1]

    e_pad = _ceil_to(n_edges, 256)
    te = 2048 if e_pad % 4096 == 0 else e_pad // 2
    tec = te
    n_pad = _ceil_to(n_nodes, 8)
    tn = 512 if n_pad % 512 == 0 else n_pad
    nb = _ceil_to(num_graphs, 8)

    srcf = jnp.full((e_pad, 1), -1.0, jnp.float32).at[:n_edges, 0].set(
        edge_index[0].astype(jnp.float32))
    tgtf = jnp.full((1, e_pad), -1.0, jnp.float32).at[0, :n_edges].set(
        edge_index[1].astype(jnp.float32))
    batchf = jnp.full((1, n_pad), -1.0, jnp.float32).at[0, :n_nodes].set(
        batch.astype(jnp.float32))

    ea8 = jnp.zeros((e_pad, 8), jnp.float32).at[:n_edges, :4].set(
        edge_attr.astype(jnp.float32))
    x_pad = jnp.zeros((n_pad, fdim), jnp.float32).at[:n_nodes].set(x)
    x8 = jnp.zeros((n_pad, 8), jnp.float32).at[:n_nodes, :4].set(x)

    def fwd(ea_l, src_l, tgt_l):
        def conv(d_nodes, w1, b1, w2, b2, w_root, bias, out_ch):
            kdim = d_nodes.shape[1]
            w1p = jnp.zeros((8, w1.shape[1]), jnp.float32).at[
                :w1.shape[0]].set(w1.astype(jnp.float32))
            msgs, xs_src = _messages(ea_l, src_l, d_nodes,
                                     w2.astype(jnp.float32), w1p,
                                     b1.reshape(1, -1).astype(jnp.float32),
                                     out_ch, te)
            return _combine(tgt_l, msgs, xs_src, d_nodes,
                            w_root.astype(jnp.float32),
                            b2.astype(jnp.float32).reshape(kdim, out_ch),
                            bias.reshape(1, -1).astype(jnp.float32), tn, tec)

        c1 = conv(x_pad, conv1_nn_w1, conv1_nn_b1, conv1_nn_w2, conv1_nn_b2,
                  conv1_root_w, conv1_bias, 256)
        d1 = jnp.concatenate([c1, x_pad], axis=1)
        c2 = conv(d1, conv2_nn_w1, conv2_nn_b1, conv2_nn_w2, conv2_nn_b2,
                  conv2_root_w, conv2_bias, 256)
        d2 = jnp.concatenate([c2, x_pad], axis=1)
        c3 = conv(d2, conv3_nn_w1, conv3_nn_b1, conv3_nn_w2, conv3_nn_b2,
                  conv3_root_w, conv3_bias, 512)

        ddim = c3.shape[1]
        w1d = fc1_w[:ddim].astype(jnp.float32)
        w1x = jnp.zeros((8, fc1_w.shape[1]), jnp.float32).at[:fdim].set(
            fc1_w[ddim:].astype(jnp.float32))
        return _head(batchf, c3, x8, w1d, w1x,
                     fc1_b.reshape(1, -1).astype(jnp.float32),
                     fc2_w.astype(jnp.float32),
                     fc2_b.reshape(1, -1).astype(jnp.float32),
                     fc3_w.astype(jnp.float32),
                     fc3_b.reshape(1, -1).astype(jnp.float32), nb, tn)

    return fwd(ea8, srcf, tgtf)[:num_graphs]


# speedup vs baseline: 1.5202x; 1.0095x over previous
import functools

import jax
import jax.numpy as jnp
from jax.experimental import pallas as pl
from jax.experimental.pallas import tpu as pltpu

_SLOPE = 0.01
_KC = 4


def _ceil_to(a, b):
    return (a + b - 1) // b * b


def _leaky(v):
    return jnp.where(v >= 0, v, _SLOPE * v)


def _sigmoid(v):
    z = jnp.exp(-jnp.abs(v))
    return jnp.where(v >= 0, 1.0 / (1.0 + z), z / (1.0 + z))


def _params(dims):
    return pltpu.CompilerParams(dimension_semantics=dims,
                                vmem_limit_bytes=60 * 1024 * 1024)


def _msg_body(ea_ref, src_ref, d_ref, w1_ref, b1_ref, w2_ref,
              o_ref, oxs_ref, h_sc, xs_sc, acc_sc, *, kc, out_ch, nk, kdim):
    k = pl.program_id(1)
    te = ea_ref.shape[0]
    n_pad = d_ref.shape[0]

    @pl.when(k == 0)
    def _():
        h = jnp.dot(ea_ref[...], w1_ref[...],
                    preferred_element_type=jnp.float32) + b1_ref[...]
        h_sc[...] = _leaky(h)
        ns = 512 if n_pad % 512 == 0 else n_pad
        node = jax.lax.broadcasted_iota(jnp.int32, (te, ns), 1)
        xs = jnp.zeros((te, d_ref.shape[1]), jnp.float32)
        for i in range(n_pad // ns):
            g = (src_ref[...] == (node + i * ns).astype(jnp.float32))
            xs = xs + jnp.dot(g.astype(jnp.float32),
                              d_ref[i * ns:(i + 1) * ns, :],
                              preferred_element_type=jnp.float32)
        xs_sc[...] = xs
        oxs_ref[...] = xs.astype(oxs_ref.dtype)
        acc_sc[...] = jnp.zeros_like(acc_sc)

    rows = jax.lax.broadcasted_iota(jnp.int32, (kdim, kc), 0)
    cols = jax.lax.broadcasted_iota(jnp.int32, (kdim, kc), 1)
    sel = (rows == k * kc + cols).astype(jnp.float32)
    xsk = jnp.dot(xs_sc[...], sel, preferred_element_type=jnp.float32)

    h = h_sc[...]
    part = jnp.dot(xsk[:, 0:1] * h, w2_ref[:, 0:out_ch],
                   preferred_element_type=jnp.float32)
    for kk in range(1, kc):
        part = part + jnp.dot(
            xsk[:, kk:kk + 1] * h, w2_ref[:, kk * out_ch:(kk + 1) * out_ch],
            preferred_element_type=jnp.float32)
    acc_sc[...] += part

    @pl.when(k == nk - 1)
    def _():
        o_ref[...] = acc_sc[...].astype(o_ref.dtype)


def _messages(ea8, srcf, d_nodes, w2, w1p, b1, out_ch, te):
    e_pad = ea8.shape[0]
    n_pad, kdim = d_nodes.shape
    hd = w1p.shape[1]
    kc = _KC
    for cand in (10, 5, 4):
        if kdim % cand == 0:
            kc = cand
            break
    nk = kdim // kc
    if hd * kc * out_ch * 4 > 6 * 1024 * 1024 and te % 256 == 0:
        te = te // 2
    hdim = w1p.shape[1]
    return pl.pallas_call(
        functools.partial(_msg_body, kc=kc, out_ch=out_ch, nk=nk, kdim=kdim),
        out_shape=(jax.ShapeDtypeStruct((e_pad, out_ch), jnp.bfloat16),
                   jax.ShapeDtypeStruct((e_pad, kdim), jnp.bfloat16)),
        grid=(e_pad // te, nk),
        in_specs=[
            pl.BlockSpec((te, 8), lambda e, k: (e, 0)),
            pl.BlockSpec((te, 1), lambda e, k: (e, 0)),
            pl.BlockSpec((n_pad, kdim), lambda e, k: (0, 0)),
            pl.BlockSpec((8, hdim), lambda e, k: (0, 0)),
            pl.BlockSpec((1, hdim), lambda e, k: (0, 0)),
            pl.BlockSpec((hdim, kc * out_ch), lambda e, k: (0, k)),
        ],
        out_specs=(pl.BlockSpec((te, out_ch), lambda e, k: (e, 0)),
                   pl.BlockSpec((te, kdim), lambda e, k: (e, 0))),
        scratch_shapes=[pltpu.VMEM((te, hdim), jnp.float32),
                        pltpu.VMEM((te, kdim), jnp.float32),
                        pltpu.VMEM((te, out_ch), jnp.float32)],
        compiler_params=_params(("parallel", "arbitrary")),
    )(ea8, srcf, d_nodes, w1p, b1, w2)


def _agg_body(tgt_ref, m_ref, xs_ref, x_ref, wr_ref, b2_ref, b_ref, o_ref,
              acc_sc, sx_sc, deg_sc, *, tn):
    n = pl.program_id(0)
    e = pl.program_id(1)

    @pl.when(e == 0)
    def _():
        acc_sc[...] = jnp.zeros_like(acc_sc)
        sx_sc[...] = jnp.zeros_like(sx_sc)
        deg_sc[...] = jnp.zeros_like(deg_sc)

    tec = m_ref.shape[0]
    rows = (jax.lax.broadcasted_iota(jnp.int32, (tn, tec), 0)
            + n * tn).astype(jnp.float32)
    mask = (rows == tgt_ref[...]).astype(jnp.float32)
    mask16 = mask.astype(jnp.bfloat16)
    acc_sc[...] += jnp.dot(mask16, m_ref[...],
                           preferred_element_type=jnp.float32)
    sx_sc[...] += jnp.dot(mask16, xs_ref[...],
                          preferred_element_type=jnp.float32)
    deg_sc[...] += jnp.sum(mask, axis=1, keepdims=True)

    @pl.when(e == pl.num_programs(1) - 1)
    def _():
        agg = acc_sc[...] + jnp.dot(sx_sc[...], b2_ref[...],
                                    preferred_element_type=jnp.float32)
        root = jnp.dot(x_ref[...], wr_ref[...],
                       preferred_element_type=jnp.float32)
        inv = 1.0 / jnp.maximum(deg_sc[...], 1.0)
        o_ref[...] = _leaky(agg * inv + root + b_ref[...])


def _combine(tgtf, msgs, xs_src, x_nodes, w_root, b2_mat, bias, tn, tec):
    n_pad = x_nodes.shape[0]
    e_pad = msgs.shape[0]
    out_ch = msgs.shape[1]
    kdim = xs_src.shape[1]
    return pl.pallas_call(
        functools.partial(_agg_body, tn=tn),
        out_shape=jax.ShapeDtypeStruct((n_pad, out_ch), jnp.float32),
        grid=(n_pad // tn, e_pad // tec),
        in_specs=[
            pl.BlockSpec((1, tec), lambda n, e: (0, e)),
            pl.BlockSpec((tec, out_ch), lambda n, e: (e, 0)),
            pl.BlockSpec((tec, kdim), lambda n, e: (e, 0)),
            pl.BlockSpec((tn, kdim), lambda n, e: (n, 0)),
            pl.BlockSpec((kdim, out_ch), lambda n, e: (0, 0)),
            pl.BlockSpec((kdim, out_ch), lambda n, e: (0, 0)),
            pl.BlockSpec((1, out_ch), lambda n, e: (0, 0)),
        ],
        out_specs=pl.BlockSpec((tn, out_ch), lambda n, e: (n, 0)),
        scratch_shapes=[pltpu.VMEM((tn, out_ch), jnp.float32),
                        pltpu.VMEM((tn, kdim), jnp.float32),
                        pltpu.VMEM((tn, 1), jnp.float32)],
        compiler_params=_params(("parallel", "arbitrary")),
    )(tgtf, msgs, xs_src, x_nodes, w_root, b2_mat, bias)


def _head_body(bf_ref, d_ref, x_ref, w1d_ref, w1x_ref, b1_ref,
               w2_ref, b2_ref, w3_ref, b3_ref, o_ref,
               pd_sc, px_sc, cnt_sc, *, nb):
    n = pl.program_id(0)

    @pl.when(n == 0)
    def _():
        pd_sc[...] = jnp.zeros_like(pd_sc)
        px_sc[...] = jnp.zeros_like(px_sc)
        cnt_sc[...] = jnp.zeros_like(cnt_sc)

    tn = d_ref.shape[0]
    gids = jax.lax.broadcasted_iota(jnp.int32, (nb, tn), 0).astype(jnp.float32)
    mask = (gids == bf_ref[...]).astype(jnp.float32)
    pd_sc[...] += jnp.dot(mask, d_ref[...],
                          preferred_element_type=jnp.float32)
    px_sc[...] += jnp.dot(mask, x_ref[...],
                          preferred_element_type=jnp.float32)
    cnt_sc[...] += jnp.sum(mask, axis=1, keepdims=True)

    @pl.when(n == pl.num_programs(0) - 1)
    def _():
        inv = 1.0 / jnp.maximum(cnt_sc[...], 1.0)
        h = jnp.dot(pd_sc[...] * inv, w1d_ref[...],
                    preferred_element_type=jnp.float32) \
            + jnp.dot(px_sc[...] * inv, w1x_ref[...],
                      preferred_element_type=jnp.float32) + b1_ref[...]
        h = _leaky(h)
        h = _leaky(jnp.dot(h, w2_ref[...],
                           preferred_element_type=jnp.float32) + b2_ref[...])
        y = jnp.dot(h, w3_ref[...],
                    preferred_element_type=jnp.float32) + b3_ref[...]
        o_ref[...] = _sigmoid(y)


def _head(batchf, d3, x8, w1d, w1x, b1, w2, b2, w3, b3, nb, tn):
    n_pad, ddim = d3.shape
    h1 = w1d.shape[1]
    h2 = w2.shape[1]
    return pl.pallas_call(
        functools.partial(_head_body, nb=nb),
        out_shape=jax.ShapeDtypeStruct((nb, 1), jnp.float32),
        grid=(n_pad // tn,),
        in_specs=[
            pl.BlockSpec((1, tn), lambda n: (0, n)),
            pl.BlockSpec((tn, ddim), lambda n: (n, 0)),
            pl.BlockSpec((tn, 8), lambda n: (n, 0)),
            pl.BlockSpec((ddim, h1), lambda n: (0, 0)),
            pl.BlockSpec((8, h1), lambda n: (0, 0)),
            pl.BlockSpec((1, h1), lambda n: (0, 0)),
            pl.BlockSpec((h1, h2), lambda n: (0, 0)),
            pl.BlockSpec((1, h2), lambda n: (0, 0)),
            pl.BlockSpec((h2, 1), lambda n: (0, 0)),
            pl.BlockSpec((1, 1), lambda n: (0, 0)),
        ],
        out_specs=pl.BlockSpec((nb, 1), lambda n: (0, 0)),
        scratch_shapes=[pltpu.VMEM((nb, ddim), jnp.float32),
                        pltpu.VMEM((nb, 8), jnp.float32),
                        pltpu.VMEM((nb, 1), jnp.float32)],
        compiler_params=_params(("arbitrary",)),
    )(batchf, d3, x8, w1d, w1x, b1, w2, b2, w3, b3)


def kernel(x, edge_index, edge_attr, batch,
           conv1_nn_w1, conv1_nn_b1, conv1_nn_w2, conv1_nn_b2,
           conv1_root_w, conv1_bias,
           conv2_nn_w1, conv2_nn_b1, conv2_nn_w2, conv2_nn_b2,
           conv2_root_w, conv2_bias,
           conv3_nn_w1, conv3_nn_b1, conv3_nn_w2, conv3_nn_b2,
           conv3_root_w, conv3_bias,
           fc1_w, fc1_b, fc2_w, fc2_b, fc3_w, fc3_b):
    num_graphs = 64
    x = x.astype(jnp.float32)
    n_nodes, fdim = x.shape
    n_edges = edge_index.shape[1]

    e_pad = _ceil_to(n_edges, 256)
    te = 2048 if e_pad % 4096 == 0 else e_pad // 2
    tec = te
    n_pad = _ceil_to(n_nodes, 8)
    tn = 512 if n_pad % 512 == 0 else n_pad
    nb = _ceil_to(num_graphs, 8)

    srcf = jnp.full((e_pad, 1), -1.0, jnp.float32).at[:n_edges, 0].set(
        edge_index[0].astype(jnp.float32))
    tgtf = jnp.full((1, e_pad), -1.0, jnp.float32).at[0, :n_edges].set(
        edge_index[1].astype(jnp.float32))
    batchf = jnp.full((1, n_pad), -1.0, jnp.float32).at[0, :n_nodes].set(
        batch.astype(jnp.float32))

    ea8 = jnp.zeros((e_pad, 8), jnp.float32).at[:n_edges, :4].set(
        edge_attr.astype(jnp.float32))
    x_pad = jnp.zeros((n_pad, fdim), jnp.float32).at[:n_nodes].set(x)
    x8 = jnp.zeros((n_pad, 8), jnp.float32).at[:n_nodes, :4].set(x)

    def fwd(ea_l, src_l, tgt_l):
        def conv(d_nodes, w1, b1, w2, b2, w_root, bias, out_ch):
            kdim = d_nodes.shape[1]
            w1p = jnp.zeros((8, w1.shape[1]), jnp.float32).at[
                :w1.shape[0]].set(w1.astype(jnp.float32))
            msgs, xs_src = _messages(ea_l, src_l, d_nodes,
                                     w2.astype(jnp.float32), w1p,
                                     b1.reshape(1, -1).astype(jnp.float32),
                                     out_ch, te)
            return _combine(tgt_l, msgs, xs_src, d_nodes,
                            w_root.astype(jnp.float32),
                            b2.astype(jnp.float32).reshape(kdim, out_ch),
                            bias.reshape(1, -1).astype(jnp.float32), tn, tec)

        c1 = conv(x_pad, conv1_nn_w1, conv1_nn_b1, conv1_nn_w2, conv1_nn_b2,
                  conv1_root_w, conv1_bias, 256)
        d1 = jnp.concatenate([c1, x_pad], axis=1)
        c2 = conv(d1, conv2_nn_w1, conv2_nn_b1, conv2_nn_w2, conv2_nn_b2,
                  conv2_root_w, conv2_bias, 256)
        d2 = jnp.concatenate([c2, x_pad], axis=1)
        c3 = conv(d2, conv3_nn_w1, conv3_nn_b1, conv3_nn_w2, conv3_nn_b2,
                  conv3_root_w, conv3_bias, 512)

        ddim = c3.shape[1]
        w1d = fc1_w[:ddim].astype(jnp.float32)
        w1x = jnp.zeros((8, fc1_w.shape[1]), jnp.float32).at[:fdim].set(
            fc1_w[ddim:].astype(jnp.float32))
        return _head(batchf, c3, x8, w1d, w1x,
                     fc1_b.reshape(1, -1).astype(jnp.float32),
                     fc2_w.astype(jnp.float32),
                     fc2_b.reshape(1, -1).astype(jnp.float32),
                     fc3_w.astype(jnp.float32),
                     fc3_b.reshape(1, -1).astype(jnp.float32), nb, tn)

    return fwd(ea8, srcf, tgtf)[:num_graphs]


# speedup vs baseline: 1.5247x; 1.0029x over previous
import functools

import jax
import jax.numpy as jnp
from jax.experimental import pallas as pl
from jax.experimental.pallas import tpu as pltpu

_SLOPE = 0.01
_KC = 4


def _ceil_to(a, b):
    return (a + b - 1) // b * b


def _leaky(v):
    return jnp.where(v >= 0, v, _SLOPE * v)


def _sigmoid(v):
    z = jnp.exp(-jnp.abs(v))
    return jnp.where(v >= 0, 1.0 / (1.0 + z), z / (1.0 + z))


def _params(dims):
    return pltpu.CompilerParams(dimension_semantics=dims,
                                vmem_limit_bytes=60 * 1024 * 1024)


def _msg_body(ea_ref, src_ref, d_ref, w1_ref, b1_ref, w2_ref,
              o_ref, oxs_ref, h_sc, xs_sc, acc_sc, *, kc, out_ch, nk, kdim):
    k = pl.program_id(1)
    te = ea_ref.shape[0]
    n_pad = d_ref.shape[0]

    @pl.when(k == 0)
    def _():
        h = jnp.dot(ea_ref[...], w1_ref[...],
                    preferred_element_type=jnp.float32) + b1_ref[...]
        h_sc[...] = _leaky(h)
        ns = 512 if n_pad % 512 == 0 else n_pad
        node = jax.lax.broadcasted_iota(jnp.int32, (te, ns), 1)
        xs = jnp.zeros((te, d_ref.shape[1]), jnp.float32)
        for i in range(n_pad // ns):
            g = (src_ref[...] == (node + i * ns).astype(jnp.float32))
            xs = xs + jnp.dot(g.astype(jnp.float32),
                              d_ref[i * ns:(i + 1) * ns, :],
                              preferred_element_type=jnp.float32)
        xs_sc[...] = xs
        oxs_ref[...] = xs.astype(oxs_ref.dtype)
        acc_sc[...] = jnp.zeros_like(acc_sc)

    rows = jax.lax.broadcasted_iota(jnp.int32, (kdim, kc), 0)
    cols = jax.lax.broadcasted_iota(jnp.int32, (kdim, kc), 1)
    sel = (rows == k * kc + cols).astype(jnp.float32)
    xsk = jnp.dot(xs_sc[...], sel, preferred_element_type=jnp.float32)

    h = h_sc[...]
    part = jnp.dot(xsk[:, 0:1] * h, w2_ref[:, 0:out_ch],
                   preferred_element_type=jnp.float32)
    for kk in range(1, kc):
        part = part + jnp.dot(
            xsk[:, kk:kk + 1] * h, w2_ref[:, kk * out_ch:(kk + 1) * out_ch],
            preferred_element_type=jnp.float32)
    acc_sc[...] += part

    @pl.when(k == nk - 1)
    def _():
        o_ref[...] = acc_sc[...].astype(o_ref.dtype)


def _messages(ea8, srcf, d_nodes, w2, w1p, b1, out_ch, te):
    e_pad = ea8.shape[0]
    n_pad, kdim = d_nodes.shape
    hd = w1p.shape[1]
    kc = _KC
    for cand in (10, 5, 4):
        if kdim % cand == 0:
            kc = cand
            break
    nk = kdim // kc
    if hd * kc * out_ch * 4 > 6 * 1024 * 1024 and te % 256 == 0:
        te = te // 2
    hdim = w1p.shape[1]
    return pl.pallas_call(
        functools.partial(_msg_body, kc=kc, out_ch=out_ch, nk=nk, kdim=kdim),
        out_shape=(jax.ShapeDtypeStruct((e_pad, out_ch), jnp.bfloat16),
                   jax.ShapeDtypeStruct((e_pad, kdim), jnp.bfloat16)),
        grid=(e_pad // te, nk),
        in_specs=[
            pl.BlockSpec((te, 8), lambda e, k: (e, 0)),
            pl.BlockSpec((te, 1), lambda e, k: (e, 0)),
            pl.BlockSpec((n_pad, kdim), lambda e, k: (0, 0)),
            pl.BlockSpec((8, hdim), lambda e, k: (0, 0)),
            pl.BlockSpec((1, hdim), lambda e, k: (0, 0)),
            pl.BlockSpec((hdim, kc * out_ch), lambda e, k: (0, k)),
        ],
        out_specs=(pl.BlockSpec((te, out_ch), lambda e, k: (e, 0)),
                   pl.BlockSpec((te, kdim), lambda e, k: (e, 0))),
        scratch_shapes=[pltpu.VMEM((te, hdim), jnp.float32),
                        pltpu.VMEM((te, kdim), jnp.float32),
                        pltpu.VMEM((te, out_ch), jnp.float32)],
        compiler_params=_params(("parallel", "arbitrary")),
    )(ea8, srcf, d_nodes, w1p, b1, w2)


def _agg_body(tgt_ref, m_ref, xs_ref, x_ref, wr_ref, b2_ref, b_ref, o_ref,
              acc_sc, sx_sc, deg_sc, *, tn):
    n = pl.program_id(0)
    e = pl.program_id(1)

    @pl.when(e == 0)
    def _():
        acc_sc[...] = jnp.zeros_like(acc_sc)
        sx_sc[...] = jnp.zeros_like(sx_sc)
        deg_sc[...] = jnp.zeros_like(deg_sc)

    tec = m_ref.shape[0]
    rows = (jax.lax.broadcasted_iota(jnp.int32, (tn, tec), 0)
            + n * tn).astype(jnp.float32)
    mask = (rows == tgt_ref[...]).astype(jnp.float32)
    mask16 = mask.astype(jnp.bfloat16)
    acc_sc[...] += jnp.dot(mask16, m_ref[...],
                           preferred_element_type=jnp.float32)
    sx_sc[...] += jnp.dot(mask16, xs_ref[...],
                          preferred_element_type=jnp.float32)
    deg_sc[...] += jnp.sum(mask, axis=1, keepdims=True)

    @pl.when(e == pl.num_programs(1) - 1)
    def _():
        agg = acc_sc[...] + jnp.dot(sx_sc[...], b2_ref[...],
                                    preferred_element_type=jnp.float32)
        root = jnp.dot(x_ref[...], wr_ref[...],
                       preferred_element_type=jnp.float32)
        inv = 1.0 / jnp.maximum(deg_sc[...], 1.0)
        o_ref[...] = _leaky(agg * inv + root + b_ref[...])


def _combine(tgtf, msgs, xs_src, x_nodes, w_root, b2_mat, bias, tn, tec):
    n_pad = x_nodes.shape[0]
    e_pad = msgs.shape[0]
    out_ch = msgs.shape[1]
    kdim = xs_src.shape[1]
    return pl.pallas_call(
        functools.partial(_agg_body, tn=tn),
        out_shape=jax.ShapeDtypeStruct((n_pad, out_ch), jnp.float32),
        grid=(n_pad // tn, e_pad // tec),
        in_specs=[
            pl.BlockSpec((1, tec), lambda n, e: (0, e)),
            pl.BlockSpec((tec, out_ch), lambda n, e: (e, 0)),
            pl.BlockSpec((tec, kdim), lambda n, e: (e, 0)),
            pl.BlockSpec((tn, kdim), lambda n, e: (n, 0)),
            pl.BlockSpec((kdim, out_ch), lambda n, e: (0, 0)),
            pl.BlockSpec((kdim, out_ch), lambda n, e: (0, 0)),
            pl.BlockSpec((1, out_ch), lambda n, e: (0, 0)),
        ],
        out_specs=pl.BlockSpec((tn, out_ch), lambda n, e: (n, 0)),
        scratch_shapes=[pltpu.VMEM((tn, out_ch), jnp.float32),
                        pltpu.VMEM((tn, kdim), jnp.float32),
                        pltpu.VMEM((tn, 1), jnp.float32)],
        compiler_params=_params(("parallel", "arbitrary")),
    )(tgtf, msgs, xs_src, x_nodes, w_root, b2_mat, bias)


def _head_body(bf_ref, d_ref, x_ref, w1d_ref, w1x_ref, b1_ref,
               w2_ref, b2_ref, w3_ref, b3_ref, o_ref,
               pd_sc, px_sc, cnt_sc, *, nb):
    n = pl.program_id(0)

    @pl.when(n == 0)
    def _():
        pd_sc[...] = jnp.zeros_like(pd_sc)
        px_sc[...] = jnp.zeros_like(px_sc)
        cnt_sc[...] = jnp.zeros_like(cnt_sc)

    tn = d_ref.shape[0]
    gids = jax.lax.broadcasted_iota(jnp.int32, (nb, tn), 0).astype(jnp.float32)
    mask = (gids == bf_ref[...]).astype(jnp.float32)
    pd_sc[...] += jnp.dot(mask, d_ref[...],
                          preferred_element_type=jnp.float32)
    px_sc[...] += jnp.dot(mask, x_ref[...],
                          preferred_element_type=jnp.float32)
    cnt_sc[...] += jnp.sum(mask, axis=1, keepdims=True)

    @pl.when(n == pl.num_programs(0) - 1)
    def _():
        inv = 1.0 / jnp.maximum(cnt_sc[...], 1.0)
        h = jnp.dot(pd_sc[...] * inv, w1d_ref[...],
                    preferred_element_type=jnp.float32) \
            + jnp.dot(px_sc[...] * inv, w1x_ref[...],
                      preferred_element_type=jnp.float32) + b1_ref[...]
        h = _leaky(h)
        h = _leaky(jnp.dot(h, w2_ref[...],
                           preferred_element_type=jnp.float32) + b2_ref[...])
        y = jnp.dot(h, w3_ref[...],
                    preferred_element_type=jnp.float32) + b3_ref[...]
        o_ref[...] = _sigmoid(y)


def _head(batchf, d3, x8, w1d, w1x, b1, w2, b2, w3, b3, nb, tn):
    n_pad, ddim = d3.shape
    h1 = w1d.shape[1]
    h2 = w2.shape[1]
    return pl.pallas_call(
        functools.partial(_head_body, nb=nb),
        out_shape=jax.ShapeDtypeStruct((nb, 1), jnp.float32),
        grid=(n_pad // tn,),
        in_specs=[
            pl.BlockSpec((1, tn), lambda n: (0, n)),
            pl.BlockSpec((tn, ddim), lambda n: (n, 0)),
            pl.BlockSpec((tn, 8), lambda n: (n, 0)),
            pl.BlockSpec((ddim, h1), lambda n: (0, 0)),
            pl.BlockSpec((8, h1), lambda n: (0, 0)),
            pl.BlockSpec((1, h1), lambda n: (0, 0)),
            pl.BlockSpec((h1, h2), lambda n: (0, 0)),
            pl.BlockSpec((1, h2), lambda n: (0, 0)),
            pl.BlockSpec((h2, 1), lambda n: (0, 0)),
            pl.BlockSpec((1, 1), lambda n: (0, 0)),
        ],
        out_specs=pl.BlockSpec((nb, 1), lambda n: (0, 0)),
        scratch_shapes=[pltpu.VMEM((nb, ddim), jnp.float32),
                        pltpu.VMEM((nb, 8), jnp.float32),
                        pltpu.VMEM((nb, 1), jnp.float32)],
        compiler_params=_params(("arbitrary",)),
    )(batchf, d3, x8, w1d, w1x, b1, w2, b2, w3, b3)


def kernel(x, edge_index, edge_attr, batch,
           conv1_nn_w1, conv1_nn_b1, conv1_nn_w2, conv1_nn_b2,
           conv1_root_w, conv1_bias,
           conv2_nn_w1, conv2_nn_b1, conv2_nn_w2, conv2_nn_b2,
           conv2_root_w, conv2_bias,
           conv3_nn_w1, conv3_nn_b1, conv3_nn_w2, conv3_nn_b2,
           conv3_root_w, conv3_bias,
           fc1_w, fc1_b, fc2_w, fc2_b, fc3_w, fc3_b):
    num_graphs = 64
    x = x.astype(jnp.float32)
    n_nodes, fdim = x.shape
    n_edges = edge_index.shape[1]

    e_pad = _ceil_to(n_edges, 256)
    te = 2048 if e_pad % 4096 == 0 else e_pad // 2
    tec = e_pad // 2 if (e_pad // 2) % 128 == 0 else e_pad
    n_pad = _ceil_to(n_nodes, 8)
    tn = 512 if n_pad % 512 == 0 else n_pad
    nb = _ceil_to(num_graphs, 8)

    srcf = jnp.full((e_pad, 1), -1.0, jnp.float32).at[:n_edges, 0].set(
        edge_index[0].astype(jnp.float32))
    tgtf = jnp.full((1, e_pad), -1.0, jnp.float32).at[0, :n_edges].set(
        edge_index[1].astype(jnp.float32))
    batchf = jnp.full((1, n_pad), -1.0, jnp.float32).at[0, :n_nodes].set(
        batch.astype(jnp.float32))

    ea8 = jnp.zeros((e_pad, 8), jnp.float32).at[:n_edges, :4].set(
        edge_attr.astype(jnp.float32))
    x_pad = jnp.zeros((n_pad, fdim), jnp.float32).at[:n_nodes].set(x)
    x8 = jnp.zeros((n_pad, 8), jnp.float32).at[:n_nodes, :4].set(x)

    def fwd(ea_l, src_l, tgt_l):
        def conv(d_nodes, w1, b1, w2, b2, w_root, bias, out_ch):
            kdim = d_nodes.shape[1]
            w1p = jnp.zeros((8, w1.shape[1]), jnp.float32).at[
                :w1.shape[0]].set(w1.astype(jnp.float32))
            msgs, xs_src = _messages(ea_l, src_l, d_nodes,
                                     w2.astype(jnp.float32), w1p,
                                     b1.reshape(1, -1).astype(jnp.float32),
                                     out_ch, te)
            return _combine(tgt_l, msgs, xs_src, d_nodes,
                            w_root.astype(jnp.float32),
                            b2.astype(jnp.float32).reshape(kdim, out_ch),
                            bias.reshape(1, -1).astype(jnp.float32), tn, tec)

        c1 = conv(x_pad, conv1_nn_w1, conv1_nn_b1, conv1_nn_w2, conv1_nn_b2,
                  conv1_root_w, conv1_bias, 256)
        d1 = jnp.concatenate([c1, x_pad], axis=1)
        c2 = conv(d1, conv2_nn_w1, conv2_nn_b1, conv2_nn_w2, conv2_nn_b2,
                  conv2_root_w, conv2_bias, 256)
        d2 = jnp.concatenate([c2, x_pad], axis=1)
        c3 = conv(d2, conv3_nn_w1, conv3_nn_b1, conv3_nn_w2, conv3_nn_b2,
                  conv3_root_w, conv3_bias, 512)

        ddim = c3.shape[1]
        w1d = fc1_w[:ddim].astype(jnp.float32)
        w1x = jnp.zeros((8, fc1_w.shape[1]), jnp.float32).at[:fdim].set(
            fc1_w[ddim:].astype(jnp.float32))
        return _head(batchf, c3, x8, w1d, w1x,
                     fc1_b.reshape(1, -1).astype(jnp.float32),
                     fc2_w.astype(jnp.float32),
                     fc2_b.reshape(1, -1).astype(jnp.float32),
                     fc3_w.astype(jnp.float32),
                     fc3_b.reshape(1, -1).astype(jnp.float32), nb, tn)

    return fwd(ea8, srcf, tgtf)[:num_graphs]
